# Initial kernel scaffold; baseline (speedup 1.0000x reference)
import functools

import jax
import jax.numpy as jnp
from jax import lax
from jax.experimental import pallas as pl
from jax.experimental.pallas import tpu as pltpu
from jax.experimental.pallas import tpu_sc as plsc

N_NODES = 10000
N_EDGES = 10000
E_PAIRS = 320000
NC = 2
NS = 16
PER_SUB = E_PAIRS // NS
BATCH = 128
FULL_BATCHES = PER_SUB // BATCH
REM = PER_SUB - FULL_BATCHES * BATCH
ROWS_PER_SUB = N_NODES // NS

_f32 = jnp.float32


def _prelu(v, a):
    return jnp.where(v >= 0, v, a * v)


def _sc_mesh():
    return plsc.VectorSubcoreMesh(core_axis_name="c", subcore_axis_name="s")


def _hist_body(hei_ref, ones_ref, zeros_ref, out_ref,
               idx_v, idx_r, ones_v, ones_r, acc, sem):
    del sem
    c = lax.axis_index("c")
    s = lax.axis_index("s")
    r0 = s * ROWS_PER_SUB
    pltpu.sync_copy(zeros_ref, acc.at[pl.ds(r0, ROWS_PER_SUB)])
    pltpu.sync_copy(ones_ref, ones_v)
    pltpu.sync_copy(ones_ref.at[pl.ds(0, REM)], ones_r)
    plsc.subcore_barrier()
    base = s * PER_SUB

    def body(g, carry):
        off = base + g * BATCH
        pltpu.sync_copy(hei_ref.at[c, pl.ds(off, BATCH)], idx_v)
        pltpu.sync_copy(ones_v, acc.at[idx_v], add=True)
        return carry

    lax.fori_loop(0, FULL_BATCHES, body, 0)
    pltpu.sync_copy(hei_ref.at[c, pl.ds(base + FULL_BATCHES * BATCH, REM)], idx_r)
    pltpu.sync_copy(ones_r, acc.at[idx_r], add=True)
    plsc.subcore_barrier()
    pltpu.sync_copy(acc.at[pl.ds(r0, ROWS_PER_SUB)],
                    out_ref.at[c, pl.ds(r0, ROWS_PER_SUB)])


def _sc_hist(hei, ones16, zeros16):
    return pl.kernel(
        _hist_body,
        out_type=jax.ShapeDtypeStruct((NC, N_NODES, 16), _f32),
        mesh=_sc_mesh(),
        scratch_types=[
            pltpu.VMEM((BATCH,), jnp.int32),
            pltpu.VMEM((REM,), jnp.int32),
            pltpu.VMEM((BATCH, 16), _f32),
            pltpu.VMEM((REM, 16), _f32),
            pltpu.VMEM_SHARED((N_NODES, 16), _f32),
            pltpu.SemaphoreType.DMA,
        ],
    )(hei, ones16, zeros16)


def _segsum_body(h_ref, src2_ref, dst_ref, zeros_ref, out_ref,
                 nv, ev, rows, nv_r, ev_r, rows_r, acc, sem):
    c = lax.axis_index("c")
    s = lax.axis_index("s")
    r0 = s * ROWS_PER_SUB
    pltpu.sync_copy(zeros_ref, acc.at[pl.ds(r0, ROWS_PER_SUB)])
    plsc.subcore_barrier()
    base = s * PER_SUB

    def body(g, carry):
        off = base + g * BATCH
        pltpu.sync_copy(src2_ref.at[c, pl.ds(off, BATCH)], nv)
        pltpu.sync_copy(dst_ref.at[pl.ds(off, BATCH)], ev)
        pltpu.async_copy(h_ref.at[nv], rows, sem).wait()
        pltpu.sync_copy(rows, acc.at[ev], add=True)
        return carry

    lax.fori_loop(0, FULL_BATCHES, body, 0)
    off_r = base + FULL_BATCHES * BATCH
    pltpu.sync_copy(src2_ref.at[c, pl.ds(off_r, REM)], nv_r)
    pltpu.sync_copy(dst_ref.at[pl.ds(off_r, REM)], ev_r)
    pltpu.async_copy(h_ref.at[nv_r], rows_r, sem).wait()
    pltpu.sync_copy(rows_r, acc.at[ev_r], add=True)
    plsc.subcore_barrier()
    pltpu.sync_copy(acc.at[pl.ds(r0, ROWS_PER_SUB)],
                    out_ref.at[c, pl.ds(r0, ROWS_PER_SUB)])


def _sc_segsum(h_flat, src2, dst, zeros128):
    return pl.kernel(
        _segsum_body,
        out_type=jax.ShapeDtypeStruct((NC, N_NODES, 128), _f32),
        mesh=_sc_mesh(),
        scratch_types=[
            pltpu.VMEM((BATCH,), jnp.int32),
            pltpu.VMEM((BATCH,), jnp.int32),
            pltpu.VMEM((BATCH, 128), _f32),
            pltpu.VMEM((REM,), jnp.int32),
            pltpu.VMEM((REM,), jnp.int32),
            pltpu.VMEM((REM, 128), _f32),
            pltpu.VMEM_SHARED((N_NODES, 128), _f32),
            pltpu.SemaphoreType.DMA,
        ],
    )(h_flat, src2, dst, zeros128)


BM = 1000
GRID = N_NODES // BM


def _chunked_spec():
    return pl.BlockSpec((NC, BM, 128), lambda i: (0, i, 0))


def _w_spec(k):
    return pl.BlockSpec((k, 256), lambda i: (0, 0))


def _b_spec():
    return pl.BlockSpec((1, 256), lambda i: (0, 0))


def _hist_spec():
    return pl.BlockSpec((BM, 16), lambda i: (i, 0))


def _a_spec():
    return pl.BlockSpec((1, 1), lambda i: (0, 0), memory_space=pltpu.SMEM)


def _write_chunked(out_ref, v):
    out_ref[0] = v[:, :128]
    out_ref[1] = v[:, 128:]


def _cat(ref):
    return jnp.concatenate([ref[0], ref[1]], axis=1)


def _tc_a1_body(x_ref, w_ref, b_ref, out_ref):
    h = jnp.dot(x_ref[...], w_ref[...], preferred_element_type=_f32) + b_ref[...]
    _write_chunked(out_ref, h)


def _tc_a1(x, W, b2d):
    return pl.pallas_call(
        _tc_a1_body,
        grid=(GRID,),
        in_specs=[pl.BlockSpec((BM, 128), lambda i: (i, 0)), _w_spec(128), _b_spec()],
        out_specs=_chunked_spec(),
        out_shape=jax.ShapeDtypeStruct((NC, N_NODES, 128), _f32),
    )(x, W, b2d)


def _tc_b_body(emit_e, se_ref, h_ref, hist_ref, w_ref, b_ref, a_ref, *out_refs):
    a = a_ref[0, 0]
    hist = hist_ref[:, 0:1]
    de_inv = jnp.where(hist > 0, 1.0 / hist, 0.0)
    e_head = _prelu(de_inv * _cat(se_ref), a)
    e_tail = _prelu(_cat(h_ref), a)
    h2h = jnp.dot(e_head, w_ref[...], preferred_element_type=_f32) + b_ref[...]
    h2t = jnp.dot(e_tail, w_ref[...], preferred_element_type=_f32) + b_ref[...]
    _write_chunked(out_refs[0], h2h)
    _write_chunked(out_refs[1], h2t)
    if emit_e:
        out_refs[2][...] = e_head


def _tc_b(se, h, hist_e, W, b2d, a2d, emit_e):
    out_shapes = [jax.ShapeDtypeStruct((NC, N_NODES, 128), _f32),
                  jax.ShapeDtypeStruct((NC, N_NODES, 128), _f32)]
    out_specs = [_chunked_spec(), _chunked_spec()]
    if emit_e:
        out_shapes.append(jax.ShapeDtypeStruct((N_NODES, 256), _f32))
        out_specs.append(pl.BlockSpec((BM, 256), lambda i: (i, 0)))
    return pl.pallas_call(
        functools.partial(_tc_b_body, emit_e),
        grid=(GRID,),
        in_specs=[_chunked_spec(), _chunked_spec(), _hist_spec(),
                  _w_spec(256), _b_spec(), _a_spec()],
        out_specs=out_specs,
        out_shape=out_shapes,
    )(se, h, hist_e, W, b2d, a2d)


def _tc_c1_body(sn_ref, h2t_ref, hist_ref, w_ref, b_ref, a_ref, out_ref):
    a = a_ref[0, 0]
    dn_inv = 1.0 / (hist_ref[:, 0:1] + 1.0)
    n1 = _prelu(dn_inv * (_cat(sn_ref) + _cat(h2t_ref)), a)
    h = jnp.dot(n1, w_ref[...], preferred_element_type=_f32) + b_ref[...]
    _write_chunked(out_ref, h)


def _tc_c1(sn, h2t, hist_n, W, b2d, a2d):
    return pl.pallas_call(
        _tc_c1_body,
        grid=(GRID,),
        in_specs=[_chunked_spec(), _chunked_spec(), _hist_spec(),
                  _w_spec(256), _b_spec(), _a_spec()],
        out_specs=_chunked_spec(),
        out_shape=jax.ShapeDtypeStruct((NC, N_NODES, 128), _f32),
    )(sn, h2t, hist_n, W, b2d, a2d)


def _tc_c2_body(sn_ref, h2t_ref, hist_ref, a_ref, out_ref):
    a = a_ref[0, 0]
    dn_inv = 1.0 / (hist_ref[:, 0:1] + 1.0)
    out_ref[...] = _prelu(dn_inv * (_cat(sn_ref) + _cat(h2t_ref)), a)


def _tc_c2(sn, h2t, hist_n, a2d):
    return pl.pallas_call(
        _tc_c2_body,
        grid=(GRID,),
        in_specs=[_chunked_spec(), _chunked_spec(), _hist_spec(), _a_spec()],
        out_specs=pl.BlockSpec((BM, 256), lambda i: (i, 0)),
        out_shape=jax.ShapeDtypeStruct((N_NODES, 256), _f32),
    )(sn, h2t, hist_n, a2d)


def kernel(x, W1_n2e, b1_n2e, W1_e2n, b1_e2n, W2_n2e, b2_n2e, W2_e2n, b2_e2n,
           prelu_a, hyperedge_index, num_nodes, num_edges):
    del num_nodes, num_edges
    ni = hyperedge_index[0]
    ei = hyperedge_index[1]
    ni2 = jnp.stack([ni, ni + N_NODES])
    ei2 = jnp.stack([ei, ei + N_NODES])

    ones16 = jnp.ones((BATCH, 16), _f32)
    zeros16 = jnp.zeros((ROWS_PER_SUB, 16), _f32)
    zeros128 = jnp.zeros((ROWS_PER_SUB, 128), _f32)
    a2d = prelu_a.reshape(1, 1)

    hists = _sc_hist(hyperedge_index, ones16, zeros16)
    hist_n = hists[0]
    hist_e = hists[1]

    h1 = _tc_a1(x, W1_n2e, b1_n2e.reshape(1, 256))
    s_e1 = _sc_segsum(h1.reshape(NC * N_NODES, 128), ni2, ei, zeros128)
    h2h1, h2t1 = _tc_b(s_e1, h1, hist_e, W1_e2n, b1_e2n.reshape(1, 256), a2d,
                       emit_e=False)
    s_n1 = _sc_segsum(h2h1.reshape(NC * N_NODES, 128), ei2, ni, zeros128)
    hA2 = _tc_c1(s_n1, h2t1, hist_n, W2_n2e, b2_n2e.reshape(1, 256), a2d)
    s_e2 = _sc_segsum(hA2.reshape(NC * N_NODES, 128), ni2, ei, zeros128)
    h2h2, h2t2, e_out = _tc_b(s_e2, hA2, hist_e, W2_e2n, b2_e2n.reshape(1, 256),
                              a2d, emit_e=True)
    s_n2 = _sc_segsum(h2h2.reshape(NC * N_NODES, 128), ei2, ni, zeros128)
    n_out = _tc_c2(s_n2, h2t2, hist_n, a2d)
    return (n_out, e_out)


# speedup vs baseline: 8.3879x; 8.3879x over previous
import functools

import jax
import jax.numpy as jnp
from jax import lax
from jax.experimental import pallas as pl
from jax.experimental.pallas import tpu as pltpu
from jax.experimental.pallas import tpu_sc as plsc

N_NODES = 10000
N_EDGES = 10000
E_PAIRS = 320000
NC = 2
NS = 16
PER_SUB = E_PAIRS // NS
BATCH = 128
FULL_BATCHES = PER_SUB // BATCH
REM = PER_SUB - FULL_BATCHES * BATCH
ROWS_PER_SUB = N_NODES // NS

_f32 = jnp.float32


def _prelu(v, a):
    return jnp.where(v >= 0, v, a * v)


def _sc_mesh():
    return plsc.VectorSubcoreMesh(core_axis_name="c", subcore_axis_name="s")


def _hist_body(hei_ref, ones_ref, zeros_ref, out_ref,
               idx_v, idx_r, ones_v, ones_r, acc, sem):
    del sem
    c = lax.axis_index("c")
    s = lax.axis_index("s")
    r0 = s * ROWS_PER_SUB
    pltpu.sync_copy(zeros_ref, acc.at[pl.ds(r0, ROWS_PER_SUB)])
    pltpu.sync_copy(ones_ref, ones_v)
    pltpu.sync_copy(ones_ref.at[pl.ds(0, REM)], ones_r)
    plsc.subcore_barrier()
    base = c * E_PAIRS + s * PER_SUB

    def body(g, carry):
        off = base + g * BATCH
        pltpu.sync_copy(hei_ref.at[pl.ds(off, BATCH)], idx_v)
        pltpu.sync_copy(ones_v, acc.at[idx_v], add=True)
        return carry

    lax.fori_loop(0, FULL_BATCHES, body, 0)
    pltpu.sync_copy(hei_ref.at[pl.ds(base + FULL_BATCHES * BATCH, REM)], idx_r)
    pltpu.sync_copy(ones_r, acc.at[idx_r], add=True)
    plsc.subcore_barrier()
    pltpu.sync_copy(acc.at[pl.ds(r0, ROWS_PER_SUB)],
                    out_ref.at[c, pl.ds(r0, ROWS_PER_SUB)])


def _sc_hist(hei, ones16, zeros16):
    return pl.kernel(
        _hist_body,
        out_type=jax.ShapeDtypeStruct((NC, N_NODES, 16), _f32),
        mesh=_sc_mesh(),
        compiler_params=pltpu.CompilerParams(use_tc_tiling_on_sc=False),
        scratch_types=[
            pltpu.VMEM((BATCH,), jnp.int32),
            pltpu.VMEM((REM,), jnp.int32),
            pltpu.VMEM((BATCH, 16), _f32),
            pltpu.VMEM((REM, 16), _f32),
            pltpu.VMEM_SHARED((N_NODES, 16), _f32),
            pltpu.SemaphoreType.DMA,
        ],
    )(hei, ones16, zeros16)


def _segsum_body(h_ref, src2_ref, dst_ref, zeros_ref, out_ref,
                 nv, ev, rows, nv_r, ev_r, rows_r, acc, sem):
    c = lax.axis_index("c")
    s = lax.axis_index("s")
    r0 = s * ROWS_PER_SUB
    pltpu.sync_copy(zeros_ref, acc.at[pl.ds(r0, ROWS_PER_SUB)])
    plsc.subcore_barrier()
    base = s * PER_SUB
    src_base = c * E_PAIRS + base

    def body(g, carry):
        off = base + g * BATCH
        pltpu.sync_copy(src2_ref.at[pl.ds(src_base + g * BATCH, BATCH)], nv)
        pltpu.sync_copy(dst_ref.at[pl.ds(off, BATCH)], ev)
        pltpu.async_copy(h_ref.at[nv], rows, sem).wait()
        pltpu.sync_copy(rows, acc.at[ev], add=True)
        return carry

    lax.fori_loop(0, FULL_BATCHES, body, 0)
    off_r = base + FULL_BATCHES * BATCH
    pltpu.sync_copy(src2_ref.at[pl.ds(src_base + FULL_BATCHES * BATCH, REM)], nv_r)
    pltpu.sync_copy(dst_ref.at[pl.ds(off_r, REM)], ev_r)
    pltpu.async_copy(h_ref.at[nv_r], rows_r, sem).wait()
    pltpu.sync_copy(rows_r, acc.at[ev_r], add=True)
    plsc.subcore_barrier()
    pltpu.sync_copy(acc.at[pl.ds(r0, ROWS_PER_SUB)],
                    out_ref.at[c, pl.ds(r0, ROWS_PER_SUB)])


def _sc_segsum(h_flat, src2, dst, zeros128):
    return pl.kernel(
        _segsum_body,
        out_type=jax.ShapeDtypeStruct((NC, N_NODES, 128), _f32),
        mesh=_sc_mesh(),
        compiler_params=pltpu.CompilerParams(use_tc_tiling_on_sc=False),
        scratch_types=[
            pltpu.VMEM((BATCH,), jnp.int32),
            pltpu.VMEM((BATCH,), jnp.int32),
            pltpu.VMEM((BATCH, 128), _f32),
            pltpu.VMEM((REM,), jnp.int32),
            pltpu.VMEM((REM,), jnp.int32),
            pltpu.VMEM((REM, 128), _f32),
            pltpu.VMEM_SHARED((N_NODES, 128), _f32),
            pltpu.SemaphoreType.DMA,
        ],
    )(h_flat, src2, dst, zeros128)


BM = 1000
GRID = N_NODES // BM


def _chunked_spec():
    return pl.BlockSpec((NC, BM, 128), lambda i: (0, i, 0))


def _w_spec(k):
    return pl.BlockSpec((k, 256), lambda i: (0, 0))


def _b_spec():
    return pl.BlockSpec((1, 256), lambda i: (0, 0))


def _hist_spec():
    return pl.BlockSpec((BM, 16), lambda i: (i, 0))


def _a_spec():
    return pl.BlockSpec((1, 1), lambda i: (0, 0), memory_space=pltpu.SMEM)


def _write_chunked(out_ref, v):
    out_ref[0] = v[:, :128]
    out_ref[1] = v[:, 128:]


def _cat(ref):
    return jnp.concatenate([ref[0], ref[1]], axis=1)


def _tc_a1_body(x_ref, w_ref, b_ref, out_ref):
    h = jnp.dot(x_ref[...], w_ref[...], preferred_element_type=_f32) + b_ref[...]
    _write_chunked(out_ref, h)


def _tc_a1(x, W, b2d):
    return pl.pallas_call(
        _tc_a1_body,
        grid=(GRID,),
        in_specs=[pl.BlockSpec((BM, 128), lambda i: (i, 0)), _w_spec(128), _b_spec()],
        out_specs=_chunked_spec(),
        out_shape=jax.ShapeDtypeStruct((NC, N_NODES, 128), _f32),
    )(x, W, b2d)


def _tc_b_body(emit_e, se_ref, h_ref, hist_ref, w_ref, b_ref, a_ref, *out_refs):
    a = a_ref[0, 0]
    hist = hist_ref[:, 0:1]
    de_inv = jnp.where(hist > 0, 1.0 / hist, 0.0)
    e_head = _prelu(de_inv * _cat(se_ref), a)
    e_tail = _prelu(_cat(h_ref), a)
    h2h = jnp.dot(e_head, w_ref[...], preferred_element_type=_f32) + b_ref[...]
    h2t = jnp.dot(e_tail, w_ref[...], preferred_element_type=_f32) + b_ref[...]
    _write_chunked(out_refs[0], h2h)
    _write_chunked(out_refs[1], h2t)
    if emit_e:
        out_refs[2][...] = e_head


def _tc_b(se, h, hist_e, W, b2d, a2d, emit_e):
    out_shapes = [jax.ShapeDtypeStruct((NC, N_NODES, 128), _f32),
                  jax.ShapeDtypeStruct((NC, N_NODES, 128), _f32)]
    out_specs = [_chunked_spec(), _chunked_spec()]
    if emit_e:
        out_shapes.append(jax.ShapeDtypeStruct((N_NODES, 256), _f32))
        out_specs.append(pl.BlockSpec((BM, 256), lambda i: (i, 0)))
    return pl.pallas_call(
        functools.partial(_tc_b_body, emit_e),
        grid=(GRID,),
        in_specs=[_chunked_spec(), _chunked_spec(), _hist_spec(),
                  _w_spec(256), _b_spec(), _a_spec()],
        out_specs=out_specs,
        out_shape=out_shapes,
    )(se, h, hist_e, W, b2d, a2d)


def _tc_c1_body(sn_ref, h2t_ref, hist_ref, w_ref, b_ref, a_ref, out_ref):
    a = a_ref[0, 0]
    dn_inv = 1.0 / (hist_ref[:, 0:1] + 1.0)
    n1 = _prelu(dn_inv * (_cat(sn_ref) + _cat(h2t_ref)), a)
    h = jnp.dot(n1, w_ref[...], preferred_element_type=_f32) + b_ref[...]
    _write_chunked(out_ref, h)


def _tc_c1(sn, h2t, hist_n, W, b2d, a2d):
    return pl.pallas_call(
        _tc_c1_body,
        grid=(GRID,),
        in_specs=[_chunked_spec(), _chunked_spec(), _hist_spec(),
                  _w_spec(256), _b_spec(), _a_spec()],
        out_specs=_chunked_spec(),
        out_shape=jax.ShapeDtypeStruct((NC, N_NODES, 128), _f32),
    )(sn, h2t, hist_n, W, b2d, a2d)


def _tc_c2_body(sn_ref, h2t_ref, hist_ref, a_ref, out_ref):
    a = a_ref[0, 0]
    dn_inv = 1.0 / (hist_ref[:, 0:1] + 1.0)
    out_ref[...] = _prelu(dn_inv * (_cat(sn_ref) + _cat(h2t_ref)), a)


def _tc_c2(sn, h2t, hist_n, a2d):
    return pl.pallas_call(
        _tc_c2_body,
        grid=(GRID,),
        in_specs=[_chunked_spec(), _chunked_spec(), _hist_spec(), _a_spec()],
        out_specs=pl.BlockSpec((BM, 256), lambda i: (i, 0)),
        out_shape=jax.ShapeDtypeStruct((N_NODES, 256), _f32),
    )(sn, h2t, hist_n, a2d)


def kernel(x, W1_n2e, b1_n2e, W1_e2n, b1_e2n, W2_n2e, b2_n2e, W2_e2n, b2_e2n,
           prelu_a, hyperedge_index, num_nodes, num_edges):
    del num_nodes, num_edges
    ni = hyperedge_index[0]
    ei = hyperedge_index[1]
    ni2 = jnp.concatenate([ni, ni + N_NODES])
    ei2 = jnp.concatenate([ei, ei + N_NODES])

    ones16 = jnp.ones((BATCH, 16), _f32)
    zeros16 = jnp.zeros((ROWS_PER_SUB, 16), _f32)
    zeros128 = jnp.zeros((ROWS_PER_SUB, 128), _f32)
    a2d = prelu_a.reshape(1, 1)

    hists = _sc_hist(hyperedge_index.reshape(2 * E_PAIRS), ones16, zeros16)
    hist_n = hists[0]
    hist_e = hists[1]

    h1 = _tc_a1(x, W1_n2e, b1_n2e.reshape(1, 256))
    s_e1 = _sc_segsum(h1.reshape(NC * N_NODES, 128), ni2, ei, zeros128)
    h2h1, h2t1 = _tc_b(s_e1, h1, hist_e, W1_e2n, b1_e2n.reshape(1, 256), a2d,
                       emit_e=False)
    s_n1 = _sc_segsum(h2h1.reshape(NC * N_NODES, 128), ei2, ni, zeros128)
    hA2 = _tc_c1(s_n1, h2t1, hist_n, W2_n2e, b2_n2e.reshape(1, 256), a2d)
    s_e2 = _sc_segsum(hA2.reshape(NC * N_NODES, 128), ni2, ei, zeros128)
    h2h2, h2t2, e_out = _tc_b(s_e2, hA2, hist_e, W2_e2n, b2_e2n.reshape(1, 256),
                              a2d, emit_e=True)
    s_n2 = _sc_segsum(h2h2.reshape(NC * N_NODES, 128), ei2, ni, zeros128)
    n_out = _tc_c2(s_n2, h2t2, hist_n, a2d)
    return (n_out, e_out)


# speedup vs baseline: 14.8974x; 1.7761x over previous
import functools

import jax
import jax.numpy as jnp
from jax import lax
from jax.experimental import pallas as pl
from jax.experimental.pallas import tpu as pltpu
from jax.experimental.pallas import tpu_sc as plsc

N_NODES = 10000
N_EDGES = 10000
E_PAIRS = 320000
NC = 2
NS = 16
PER_SUB = E_PAIRS // NS
BATCH = 128
FULL_BATCHES = PER_SUB // BATCH
REM = PER_SUB - FULL_BATCHES * BATCH
ROWS_PER_SUB = N_NODES // NS

_f32 = jnp.float32


def _prelu(v, a):
    return jnp.where(v >= 0, v, a * v)


def _sc_mesh():
    return plsc.VectorSubcoreMesh(core_axis_name="c", subcore_axis_name="s")


def _hist_body(hei_ref, ones_ref, zeros_ref, out_ref,
               idx_v, idx_r, ones_v, ones_r, acc, sem):
    del sem
    c = lax.axis_index("c")
    s = lax.axis_index("s")
    r0 = s * ROWS_PER_SUB
    pltpu.sync_copy(zeros_ref, acc.at[pl.ds(r0, ROWS_PER_SUB)])
    pltpu.sync_copy(ones_ref, ones_v)
    pltpu.sync_copy(ones_ref.at[pl.ds(0, REM)], ones_r)
    plsc.subcore_barrier()
    base = c * E_PAIRS + s * PER_SUB

    def body(g, carry):
        off = base + g * BATCH
        pltpu.sync_copy(hei_ref.at[pl.ds(off, BATCH)], idx_v)
        pltpu.sync_copy(ones_v, acc.at[idx_v], add=True)
        return carry

    lax.fori_loop(0, FULL_BATCHES, body, 0)
    pltpu.sync_copy(hei_ref.at[pl.ds(base + FULL_BATCHES * BATCH, REM)], idx_r)
    pltpu.sync_copy(ones_r, acc.at[idx_r], add=True)
    plsc.subcore_barrier()
    pltpu.sync_copy(acc.at[pl.ds(r0, ROWS_PER_SUB)],
                    out_ref.at[c, pl.ds(r0, ROWS_PER_SUB)])


def _sc_hist(hei, ones16, zeros16):
    return pl.kernel(
        _hist_body,
        out_type=jax.ShapeDtypeStruct((NC, N_NODES, 16), _f32),
        mesh=_sc_mesh(),
        compiler_params=pltpu.CompilerParams(use_tc_tiling_on_sc=False),
        scratch_types=[
            pltpu.VMEM((BATCH,), jnp.int32),
            pltpu.VMEM((REM,), jnp.int32),
            pltpu.VMEM((BATCH, 16), _f32),
            pltpu.VMEM((REM, 16), _f32),
            pltpu.VMEM_SHARED((N_NODES, 16), _f32),
            pltpu.SemaphoreType.DMA,
        ],
    )(hei, ones16, zeros16)


def _segsum_body(h_ref, src2_ref, dst_ref, zeros_ref, out_ref,
                 nv2, ev2, rows2, nv_r, ev_r, rows_r, acc, sem_g, sem_i):
    c = lax.axis_index("c")
    s = lax.axis_index("s")
    r0 = s * ROWS_PER_SUB
    pltpu.sync_copy(zeros_ref, acc.at[pl.ds(r0, ROWS_PER_SUB)])
    plsc.subcore_barrier()
    base = s * PER_SUB
    src_base = c * E_PAIRS + base

    def idx_start(g, slot):
        pltpu.async_copy(src2_ref.at[pl.ds(src_base + g * BATCH, BATCH)],
                         nv2.at[slot], sem_i)
        pltpu.async_copy(dst_ref.at[pl.ds(base + g * BATCH, BATCH)],
                         ev2.at[slot], sem_i)

    def idx_wait(slot):
        pltpu.make_async_copy(src2_ref.at[pl.ds(0, BATCH)], nv2.at[slot],
                              sem_i).wait()
        pltpu.make_async_copy(dst_ref.at[pl.ds(0, BATCH)], ev2.at[slot],
                              sem_i).wait()

    idx_start(0, 0)
    idx_wait(0)
    pltpu.async_copy(h_ref.at[nv2.at[0]], rows2.at[0], sem_g)
    idx_start(1, 1)

    def body(g, carry):
        p = lax.rem(g, 2)
        q = 1 - p
        pltpu.make_async_copy(h_ref.at[nv2.at[p]], rows2.at[p], sem_g).wait()

        @pl.when(g + 1 < FULL_BATCHES)
        def _():
            idx_wait(q)
            pltpu.async_copy(h_ref.at[nv2.at[q]], rows2.at[q], sem_g)

        pltpu.sync_copy(rows2.at[p], acc.at[ev2.at[p]], add=True)

        @pl.when(g + 2 < FULL_BATCHES)
        def _():
            idx_start(g + 2, p)

        return carry

    lax.fori_loop(0, FULL_BATCHES, body, 0)
    off_r = base + FULL_BATCHES * BATCH
    pltpu.sync_copy(src2_ref.at[pl.ds(src_base + FULL_BATCHES * BATCH, REM)], nv_r)
    pltpu.sync_copy(dst_ref.at[pl.ds(off_r, REM)], ev_r)
    pltpu.async_copy(h_ref.at[nv_r], rows_r, sem_g).wait()
    pltpu.sync_copy(rows_r, acc.at[ev_r], add=True)
    plsc.subcore_barrier()
    pltpu.sync_copy(acc.at[pl.ds(r0, ROWS_PER_SUB)],
                    out_ref.at[c, pl.ds(r0, ROWS_PER_SUB)])


def _sc_segsum(h_flat, src2, dst, zeros128):
    return pl.kernel(
        _segsum_body,
        out_type=jax.ShapeDtypeStruct((NC, N_NODES, 128), _f32),
        mesh=_sc_mesh(),
        compiler_params=pltpu.CompilerParams(use_tc_tiling_on_sc=False),
        scratch_types=[
            pltpu.VMEM((2, BATCH), jnp.int32),
            pltpu.VMEM((2, BATCH), jnp.int32),
            pltpu.VMEM((2, BATCH, 128), _f32),
            pltpu.VMEM((REM,), jnp.int32),
            pltpu.VMEM((REM,), jnp.int32),
            pltpu.VMEM((REM, 128), _f32),
            pltpu.VMEM_SHARED((N_NODES, 128), _f32),
            pltpu.SemaphoreType.DMA,
            pltpu.SemaphoreType.DMA,
        ],
    )(h_flat, src2, dst, zeros128)


BM = 1000
GRID = N_NODES // BM


def _chunked_spec():
    return pl.BlockSpec((NC, BM, 128), lambda i: (0, i, 0))


def _w_spec(k):
    return pl.BlockSpec((k, 256), lambda i: (0, 0))


def _b_spec():
    return pl.BlockSpec((1, 256), lambda i: (0, 0))


def _hist_spec():
    return pl.BlockSpec((BM, 16), lambda i: (i, 0))


def _a_spec():
    return pl.BlockSpec((1, 1), lambda i: (0, 0), memory_space=pltpu.SMEM)


def _write_chunked(out_ref, v):
    out_ref[0] = v[:, :128]
    out_ref[1] = v[:, 128:]


def _cat(ref):
    return jnp.concatenate([ref[0], ref[1]], axis=1)


def _tc_a1_body(x_ref, w_ref, b_ref, out_ref):
    h = jnp.dot(x_ref[...], w_ref[...], preferred_element_type=_f32) + b_ref[...]
    _write_chunked(out_ref, h)


def _tc_a1(x, W, b2d):
    return pl.pallas_call(
        _tc_a1_body,
        grid=(GRID,),
        in_specs=[pl.BlockSpec((BM, 128), lambda i: (i, 0)), _w_spec(128), _b_spec()],
        out_specs=_chunked_spec(),
        out_shape=jax.ShapeDtypeStruct((NC, N_NODES, 128), _f32),
    )(x, W, b2d)


def _tc_b_body(emit_e, se_ref, h_ref, hist_ref, w_ref, b_ref, a_ref, *out_refs):
    a = a_ref[0, 0]
    hist = hist_ref[:, 0:1]
    de_inv = jnp.where(hist > 0, 1.0 / hist, 0.0)
    e_head = _prelu(de_inv * _cat(se_ref), a)
    e_tail = _prelu(_cat(h_ref), a)
    h2h = jnp.dot(e_head, w_ref[...], preferred_element_type=_f32) + b_ref[...]
    h2t = jnp.dot(e_tail, w_ref[...], preferred_element_type=_f32) + b_ref[...]
    _write_chunked(out_refs[0], h2h)
    _write_chunked(out_refs[1], h2t)
    if emit_e:
        out_refs[2][...] = e_head


def _tc_b(se, h, hist_e, W, b2d, a2d, emit_e):
    out_shapes = [jax.ShapeDtypeStruct((NC, N_NODES, 128), _f32),
                  jax.ShapeDtypeStruct((NC, N_NODES, 128), _f32)]
    out_specs = [_chunked_spec(), _chunked_spec()]
    if emit_e:
        out_shapes.append(jax.ShapeDtypeStruct((N_NODES, 256), _f32))
        out_specs.append(pl.BlockSpec((BM, 256), lambda i: (i, 0)))
    return pl.pallas_call(
        functools.partial(_tc_b_body, emit_e),
        grid=(GRID,),
        in_specs=[_chunked_spec(), _chunked_spec(), _hist_spec(),
                  _w_spec(256), _b_spec(), _a_spec()],
        out_specs=out_specs,
        out_shape=out_shapes,
    )(se, h, hist_e, W, b2d, a2d)


def _tc_c1_body(sn_ref, h2t_ref, hist_ref, w_ref, b_ref, a_ref, out_ref):
    a = a_ref[0, 0]
    dn_inv = 1.0 / (hist_ref[:, 0:1] + 1.0)
    n1 = _prelu(dn_inv * (_cat(sn_ref) + _cat(h2t_ref)), a)
    h = jnp.dot(n1, w_ref[...], preferred_element_type=_f32) + b_ref[...]
    _write_chunked(out_ref, h)


def _tc_c1(sn, h2t, hist_n, W, b2d, a2d):
    return pl.pallas_call(
        _tc_c1_body,
        grid=(GRID,),
        in_specs=[_chunked_spec(), _chunked_spec(), _hist_spec(),
                  _w_spec(256), _b_spec(), _a_spec()],
        out_specs=_chunked_spec(),
        out_shape=jax.ShapeDtypeStruct((NC, N_NODES, 128), _f32),
    )(sn, h2t, hist_n, W, b2d, a2d)


def _tc_c2_body(sn_ref, h2t_ref, hist_ref, a_ref, out_ref):
    a = a_ref[0, 0]
    dn_inv = 1.0 / (hist_ref[:, 0:1] + 1.0)
    out_ref[...] = _prelu(dn_inv * (_cat(sn_ref) + _cat(h2t_ref)), a)


def _tc_c2(sn, h2t, hist_n, a2d):
    return pl.pallas_call(
        _tc_c2_body,
        grid=(GRID,),
        in_specs=[_chunked_spec(), _chunked_spec(), _hist_spec(), _a_spec()],
        out_specs=pl.BlockSpec((BM, 256), lambda i: (i, 0)),
        out_shape=jax.ShapeDtypeStruct((N_NODES, 256), _f32),
    )(sn, h2t, hist_n, a2d)


def kernel(x, W1_n2e, b1_n2e, W1_e2n, b1_e2n, W2_n2e, b2_n2e, W2_e2n, b2_e2n,
           prelu_a, hyperedge_index, num_nodes, num_edges):
    del num_nodes, num_edges
    ni = hyperedge_index[0]
    ei = hyperedge_index[1]
    ni2 = jnp.concatenate([ni, ni + N_NODES])
    ei2 = jnp.concatenate([ei, ei + N_NODES])

    ones16 = jnp.ones((BATCH, 16), _f32)
    zeros16 = jnp.zeros((ROWS_PER_SUB, 16), _f32)
    zeros128 = jnp.zeros((ROWS_PER_SUB, 128), _f32)
    a2d = prelu_a.reshape(1, 1)

    hists = _sc_hist(hyperedge_index.reshape(2 * E_PAIRS), ones16, zeros16)
    hist_n = hists[0]
    hist_e = hists[1]

    h1 = _tc_a1(x, W1_n2e, b1_n2e.reshape(1, 256))
    s_e1 = _sc_segsum(h1.reshape(NC * N_NODES, 128), ni2, ei, zeros128)
    h2h1, h2t1 = _tc_b(s_e1, h1, hist_e, W1_e2n, b1_e2n.reshape(1, 256), a2d,
                       emit_e=False)
    s_n1 = _sc_segsum(h2h1.reshape(NC * N_NODES, 128), ei2, ni, zeros128)
    hA2 = _tc_c1(s_n1, h2t1, hist_n, W2_n2e, b2_n2e.reshape(1, 256), a2d)
    s_e2 = _sc_segsum(hA2.reshape(NC * N_NODES, 128), ni2, ei, zeros128)
    h2h2, h2t2, e_out = _tc_b(s_e2, hA2, hist_e, W2_e2n, b2_e2n.reshape(1, 256),
                              a2d, emit_e=True)
    s_n2 = _sc_segsum(h2h2.reshape(NC * N_NODES, 128), ei2, ni, zeros128)
    n_out = _tc_c2(s_n2, h2t2, hist_n, a2d)
    return (n_out, e_out)


# speedup vs baseline: 14.9259x; 1.0019x over previous
import functools

import jax
import jax.numpy as jnp
from jax import lax
from jax.experimental import pallas as pl
from jax.experimental.pallas import tpu as pltpu
from jax.experimental.pallas import tpu_sc as plsc

N_NODES = 10000
N_EDGES = 10000
E_PAIRS = 320000
NC = 2
NS = 16
PER_SUB = E_PAIRS // NS
BATCH = 128
FULL_BATCHES = PER_SUB // BATCH
REM = PER_SUB - FULL_BATCHES * BATCH
ROWS_PER_SUB = N_NODES // NS

_f32 = jnp.float32


def _prelu(v, a):
    return jnp.where(v >= 0, v, a * v)


def _sc_mesh():
    return plsc.VectorSubcoreMesh(core_axis_name="c", subcore_axis_name="s")


def _hist_body(hei_ref, ones_ref, zeros_ref, out_ref,
               idx_v, idx_r, ones_v, ones_r, acc, sem):
    del sem
    c = lax.axis_index("c")
    s = lax.axis_index("s")
    r0 = s * ROWS_PER_SUB
    pltpu.sync_copy(zeros_ref, acc.at[pl.ds(r0, ROWS_PER_SUB)])
    pltpu.sync_copy(ones_ref, ones_v)
    pltpu.sync_copy(ones_ref.at[pl.ds(0, REM)], ones_r)
    plsc.subcore_barrier()
    base = c * E_PAIRS + s * PER_SUB

    def body(g, carry):
        off = base + g * BATCH
        pltpu.sync_copy(hei_ref.at[pl.ds(off, BATCH)], idx_v)
        pltpu.sync_copy(ones_v, acc.at[idx_v], add=True)
        return carry

    lax.fori_loop(0, FULL_BATCHES, body, 0)
    pltpu.sync_copy(hei_ref.at[pl.ds(base + FULL_BATCHES * BATCH, REM)], idx_r)
    pltpu.sync_copy(ones_r, acc.at[idx_r], add=True)
    plsc.subcore_barrier()
    pltpu.sync_copy(acc.at[pl.ds(r0, ROWS_PER_SUB)],
                    out_ref.at[c, pl.ds(r0, ROWS_PER_SUB)])


def _sc_hist(hei, ones16, zeros16):
    return pl.kernel(
        _hist_body,
        out_type=jax.ShapeDtypeStruct((NC, N_NODES, 16), _f32),
        mesh=_sc_mesh(),
        compiler_params=pltpu.CompilerParams(use_tc_tiling_on_sc=False),
        scratch_types=[
            pltpu.VMEM((BATCH,), jnp.int32),
            pltpu.VMEM((REM,), jnp.int32),
            pltpu.VMEM((BATCH, 16), _f32),
            pltpu.VMEM((REM, 16), _f32),
            pltpu.VMEM_SHARED((N_NODES, 16), _f32),
            pltpu.SemaphoreType.DMA,
        ],
    )(hei, ones16, zeros16)


def _segsum_body(h_ref, src2_ref, dst_ref, zeros_ref, out_ref,
                 nv2, ev2, rows2, nv_r, ev_r, rows_r, acc, sem_g, sem_i, sem_s):
    c = lax.axis_index("c")
    s = lax.axis_index("s")
    r0 = s * ROWS_PER_SUB
    pltpu.sync_copy(zeros_ref, acc.at[pl.ds(r0, ROWS_PER_SUB)])
    plsc.subcore_barrier()
    base = s * PER_SUB
    src_base = c * E_PAIRS + base

    def idx_start(g, slot):
        pltpu.async_copy(src2_ref.at[pl.ds(src_base + g * BATCH, BATCH)],
                         nv2.at[slot], sem_i)
        pltpu.async_copy(dst_ref.at[pl.ds(base + g * BATCH, BATCH)],
                         ev2.at[slot], sem_i)

    def idx_wait(slot):
        pltpu.make_async_copy(src2_ref.at[pl.ds(0, BATCH)], nv2.at[slot],
                              sem_i).wait()
        pltpu.make_async_copy(dst_ref.at[pl.ds(0, BATCH)], ev2.at[slot],
                              sem_i).wait()

    def scat_wait(p):
        pltpu.make_async_copy(rows2.at[p], acc.at[ev2.at[0]], sem_s).wait()

    idx_start(0, 0)
    idx_wait(0)
    pltpu.async_copy(h_ref.at[nv2.at[0]], rows2.at[0], sem_g)
    idx_start(1, 1)

    def body(g, carry):
        p = lax.rem(g, 2)
        q = 1 - p
        t = lax.rem(g, 3)
        pltpu.make_async_copy(h_ref.at[nv2.at[p]], rows2.at[p], sem_g).wait()

        @pl.when(g >= 1)
        def _():
            scat_wait(q)

        @pl.when(g + 1 < FULL_BATCHES)
        def _():
            idx_wait(lax.rem(g + 1, 3))
            pltpu.async_copy(h_ref.at[nv2.at[lax.rem(g + 1, 3)]],
                             rows2.at[q], sem_g)

        pltpu.async_copy(rows2.at[p], acc.at[ev2.at[t]], sem_s, add=True)

        @pl.when(g + 2 < FULL_BATCHES)
        def _():
            idx_start(g + 2, lax.rem(g + 2, 3))

        return carry

    lax.fori_loop(0, FULL_BATCHES, body, 0)
    scat_wait(lax.rem(FULL_BATCHES - 1, 2))
    off_r = base + FULL_BATCHES * BATCH
    pltpu.sync_copy(src2_ref.at[pl.ds(src_base + FULL_BATCHES * BATCH, REM)], nv_r)
    pltpu.sync_copy(dst_ref.at[pl.ds(off_r, REM)], ev_r)
    pltpu.async_copy(h_ref.at[nv_r], rows_r, sem_g).wait()
    pltpu.sync_copy(rows_r, acc.at[ev_r], add=True)
    plsc.subcore_barrier()
    pltpu.sync_copy(acc.at[pl.ds(r0, ROWS_PER_SUB)],
                    out_ref.at[c, pl.ds(r0, ROWS_PER_SUB)])


def _sc_segsum(h_flat, src2, dst, zeros128):
    return pl.kernel(
        _segsum_body,
        out_type=jax.ShapeDtypeStruct((NC, N_NODES, 128), _f32),
        mesh=_sc_mesh(),
        compiler_params=pltpu.CompilerParams(use_tc_tiling_on_sc=False),
        scratch_types=[
            pltpu.VMEM((3, BATCH), jnp.int32),
            pltpu.VMEM((3, BATCH), jnp.int32),
            pltpu.VMEM((2, BATCH, 128), _f32),
            pltpu.VMEM((REM,), jnp.int32),
            pltpu.VMEM((REM,), jnp.int32),
            pltpu.VMEM((REM, 128), _f32),
            pltpu.VMEM_SHARED((N_NODES, 128), _f32),
            pltpu.SemaphoreType.DMA,
            pltpu.SemaphoreType.DMA,
            pltpu.SemaphoreType.DMA,
        ],
    )(h_flat, src2, dst, zeros128)


BM = 1000
GRID = N_NODES // BM


def _chunked_spec():
    return pl.BlockSpec((NC, BM, 128), lambda i: (0, i, 0))


def _w_spec(k):
    return pl.BlockSpec((k, 256), lambda i: (0, 0))


def _b_spec():
    return pl.BlockSpec((1, 256), lambda i: (0, 0))


def _hist_spec():
    return pl.BlockSpec((BM, 16), lambda i: (i, 0))


def _a_spec():
    return pl.BlockSpec((1, 1), lambda i: (0, 0), memory_space=pltpu.SMEM)


def _write_chunked(out_ref, v):
    out_ref[0] = v[:, :128]
    out_ref[1] = v[:, 128:]


def _cat(ref):
    return jnp.concatenate([ref[0], ref[1]], axis=1)


def _tc_a1_body(x_ref, w_ref, b_ref, out_ref):
    h = jnp.dot(x_ref[...], w_ref[...], preferred_element_type=_f32) + b_ref[...]
    _write_chunked(out_ref, h)


def _tc_a1(x, W, b2d):
    return pl.pallas_call(
        _tc_a1_body,
        grid=(GRID,),
        in_specs=[pl.BlockSpec((BM, 128), lambda i: (i, 0)), _w_spec(128), _b_spec()],
        out_specs=_chunked_spec(),
        out_shape=jax.ShapeDtypeStruct((NC, N_NODES, 128), _f32),
    )(x, W, b2d)


def _tc_b_body(emit_e, se_ref, h_ref, hist_ref, w_ref, b_ref, a_ref, *out_refs):
    a = a_ref[0, 0]
    hist = hist_ref[:, 0:1]
    de_inv = jnp.where(hist > 0, 1.0 / hist, 0.0)
    e_head = _prelu(de_inv * _cat(se_ref), a)
    e_tail = _prelu(_cat(h_ref), a)
    h2h = jnp.dot(e_head, w_ref[...], preferred_element_type=_f32) + b_ref[...]
    h2t = jnp.dot(e_tail, w_ref[...], preferred_element_type=_f32) + b_ref[...]
    _write_chunked(out_refs[0], h2h)
    _write_chunked(out_refs[1], h2t)
    if emit_e:
        out_refs[2][...] = e_head


def _tc_b(se, h, hist_e, W, b2d, a2d, emit_e):
    out_shapes = [jax.ShapeDtypeStruct((NC, N_NODES, 128), _f32),
                  jax.ShapeDtypeStruct((NC, N_NODES, 128), _f32)]
    out_specs = [_chunked_spec(), _chunked_spec()]
    if emit_e:
        out_shapes.append(jax.ShapeDtypeStruct((N_NODES, 256), _f32))
        out_specs.append(pl.BlockSpec((BM, 256), lambda i: (i, 0)))
    return pl.pallas_call(
        functools.partial(_tc_b_body, emit_e),
        grid=(GRID,),
        in_specs=[_chunked_spec(), _chunked_spec(), _hist_spec(),
                  _w_spec(256), _b_spec(), _a_spec()],
        out_specs=out_specs,
        out_shape=out_shapes,
    )(se, h, hist_e, W, b2d, a2d)


def _tc_c1_body(sn_ref, h2t_ref, hist_ref, w_ref, b_ref, a_ref, out_ref):
    a = a_ref[0, 0]
    dn_inv = 1.0 / (hist_ref[:, 0:1] + 1.0)
    n1 = _prelu(dn_inv * (_cat(sn_ref) + _cat(h2t_ref)), a)
    h = jnp.dot(n1, w_ref[...], preferred_element_type=_f32) + b_ref[...]
    _write_chunked(out_ref, h)


def _tc_c1(sn, h2t, hist_n, W, b2d, a2d):
    return pl.pallas_call(
        _tc_c1_body,
        grid=(GRID,),
        in_specs=[_chunked_spec(), _chunked_spec(), _hist_spec(),
                  _w_spec(256), _b_spec(), _a_spec()],
        out_specs=_chunked_spec(),
        out_shape=jax.ShapeDtypeStruct((NC, N_NODES, 128), _f32),
    )(sn, h2t, hist_n, W, b2d, a2d)


def _tc_c2_body(sn_ref, h2t_ref, hist_ref, a_ref, out_ref):
    a = a_ref[0, 0]
    dn_inv = 1.0 / (hist_ref[:, 0:1] + 1.0)
    out_ref[...] = _prelu(dn_inv * (_cat(sn_ref) + _cat(h2t_ref)), a)


def _tc_c2(sn, h2t, hist_n, a2d):
    return pl.pallas_call(
        _tc_c2_body,
        grid=(GRID,),
        in_specs=[_chunked_spec(), _chunked_spec(), _hist_spec(), _a_spec()],
        out_specs=pl.BlockSpec((BM, 256), lambda i: (i, 0)),
        out_shape=jax.ShapeDtypeStruct((N_NODES, 256), _f32),
    )(sn, h2t, hist_n, a2d)


def kernel(x, W1_n2e, b1_n2e, W1_e2n, b1_e2n, W2_n2e, b2_n2e, W2_e2n, b2_e2n,
           prelu_a, hyperedge_index, num_nodes, num_edges):
    del num_nodes, num_edges
    ni = hyperedge_index[0]
    ei = hyperedge_index[1]
    ni2 = jnp.concatenate([ni, ni + N_NODES])
    ei2 = jnp.concatenate([ei, ei + N_NODES])

    ones16 = jnp.ones((BATCH, 16), _f32)
    zeros16 = jnp.zeros((ROWS_PER_SUB, 16), _f32)
    zeros128 = jnp.zeros((ROWS_PER_SUB, 128), _f32)
    a2d = prelu_a.reshape(1, 1)

    hists = _sc_hist(hyperedge_index.reshape(2 * E_PAIRS), ones16, zeros16)
    hist_n = hists[0]
    hist_e = hists[1]

    h1 = _tc_a1(x, W1_n2e, b1_n2e.reshape(1, 256))
    s_e1 = _sc_segsum(h1.reshape(NC * N_NODES, 128), ni2, ei, zeros128)
    h2h1, h2t1 = _tc_b(s_e1, h1, hist_e, W1_e2n, b1_e2n.reshape(1, 256), a2d,
                       emit_e=False)
    s_n1 = _sc_segsum(h2h1.reshape(NC * N_NODES, 128), ei2, ni, zeros128)
    hA2 = _tc_c1(s_n1, h2t1, hist_n, W2_n2e, b2_n2e.reshape(1, 256), a2d)
    s_e2 = _sc_segsum(hA2.reshape(NC * N_NODES, 128), ni2, ei, zeros128)
    h2h2, h2t2, e_out = _tc_b(s_e2, hA2, hist_e, W2_e2n, b2_e2n.reshape(1, 256),
                              a2d, emit_e=True)
    s_n2 = _sc_segsum(h2h2.reshape(NC * N_NODES, 128), ei2, ni, zeros128)
    n_out = _tc_c2(s_n2, h2t2, hist_n, a2d)
    return (n_out, e_out)


# speedup vs baseline: 15.5202x; 1.0398x over previous
import functools

import jax
import jax.numpy as jnp
from jax import lax
from jax.experimental import pallas as pl
from jax.experimental.pallas import tpu as pltpu
from jax.experimental.pallas import tpu_sc as plsc

N_NODES = 10000
N_EDGES = 10000
E_PAIRS = 320000
NC = 2
NS = 16
PER_SUB = E_PAIRS // NS
BATCH = 128
FULL_BATCHES = PER_SUB // BATCH
REM = PER_SUB - FULL_BATCHES * BATCH
ROWS_PER_SUB = N_NODES // NS

_f32 = jnp.float32


def _prelu(v, a):
    return jnp.where(v >= 0, v, a * v)


def _sc_mesh():
    return plsc.VectorSubcoreMesh(core_axis_name="c", subcore_axis_name="s")


def _hist_body(hei_ref, ones_ref, zeros_ref, out_ref,
               idx2, idx_r, ones_v, ones_r, acc, sem_i):
    c = lax.axis_index("c")
    s = lax.axis_index("s")
    r0 = s * ROWS_PER_SUB
    pltpu.sync_copy(zeros_ref, acc.at[pl.ds(r0, ROWS_PER_SUB)])
    pltpu.sync_copy(ones_ref, ones_v)
    pltpu.sync_copy(ones_ref.at[pl.ds(0, REM)], ones_r)
    plsc.subcore_barrier()
    base = c * E_PAIRS + s * PER_SUB

    def idx_start(g, slot):
        pltpu.async_copy(hei_ref.at[pl.ds(base + g * BATCH, BATCH)],
                         idx2.at[slot], sem_i)

    def idx_wait(slot):
        pltpu.make_async_copy(hei_ref.at[pl.ds(0, BATCH)], idx2.at[slot],
                              sem_i).wait()

    idx_start(0, 0)
    idx_start(1, 1)

    def body(g, carry):
        p = lax.rem(g, 2)
        idx_wait(p)
        pltpu.sync_copy(ones_v, acc.at[idx2.at[p]], add=True)

        @pl.when(g + 2 < FULL_BATCHES)
        def _():
            idx_start(g + 2, p)

        return carry

    lax.fori_loop(0, FULL_BATCHES, body, 0)
    pltpu.sync_copy(hei_ref.at[pl.ds(base + FULL_BATCHES * BATCH, REM)], idx_r)
    pltpu.sync_copy(ones_r, acc.at[idx_r], add=True)
    plsc.subcore_barrier()
    pltpu.sync_copy(acc.at[pl.ds(r0, ROWS_PER_SUB)],
                    out_ref.at[c, pl.ds(r0, ROWS_PER_SUB)])


def _sc_hist(hei, ones16, zeros16):
    return pl.kernel(
        _hist_body,
        out_type=jax.ShapeDtypeStruct((NC, N_NODES, 16), _f32),
        mesh=_sc_mesh(),
        compiler_params=pltpu.CompilerParams(use_tc_tiling_on_sc=False),
        scratch_types=[
            pltpu.VMEM((2, BATCH), jnp.int32),
            pltpu.VMEM((REM,), jnp.int32),
            pltpu.VMEM((BATCH, 16), _f32),
            pltpu.VMEM((REM, 16), _f32),
            pltpu.VMEM_SHARED((N_NODES, 16), _f32),
            pltpu.SemaphoreType.DMA,
        ],
    )(hei, ones16, zeros16)


def _segsum_body(h_ref, src2_ref, dst_ref, zeros_ref, out_ref,
                 nv2, ev2, rows2, nv_r, ev_r, rows_r, acc, sem_g, sem_i, sem_s):
    c = lax.axis_index("c")
    s = lax.axis_index("s")
    r0 = s * ROWS_PER_SUB
    pltpu.sync_copy(zeros_ref, acc.at[pl.ds(r0, ROWS_PER_SUB)])
    plsc.subcore_barrier()
    base = s * PER_SUB
    src_base = c * E_PAIRS + base

    def idx_start(g, slot):
        pltpu.async_copy(src2_ref.at[pl.ds(src_base + g * BATCH, BATCH)],
                         nv2.at[slot], sem_i)
        pltpu.async_copy(dst_ref.at[pl.ds(base + g * BATCH, BATCH)],
                         ev2.at[slot], sem_i)

    def idx_wait(slot):
        pltpu.make_async_copy(src2_ref.at[pl.ds(0, BATCH)], nv2.at[slot],
                              sem_i).wait()
        pltpu.make_async_copy(dst_ref.at[pl.ds(0, BATCH)], ev2.at[slot],
                              sem_i).wait()

    def scat_wait():
        pltpu.make_async_copy(rows2.at[0], acc.at[ev2.at[0]], sem_s).wait()

    idx_start(0, 0)
    idx_wait(0)
    pltpu.async_copy(h_ref.at[nv2.at[0]], rows2.at[0], sem_g)
    idx_start(1, 1)

    def body(g, carry):
        p = lax.rem(g, 2)
        q = 1 - p
        t = lax.rem(g, 3)
        pltpu.make_async_copy(h_ref.at[nv2.at[p]], rows2.at[p], sem_g).wait()

        @pl.when(g >= 1)
        def _():
            scat_wait()

        @pl.when(g + 1 < FULL_BATCHES)
        def _():
            t1 = lax.rem(g + 1, 3)
            idx_wait(t1)
            pltpu.async_copy(h_ref.at[nv2.at[t1]], rows2.at[q], sem_g)

        pltpu.async_copy(rows2.at[p], acc.at[ev2.at[t]], sem_s, add=True)

        @pl.when(g + 2 < FULL_BATCHES)
        def _():
            idx_start(g + 2, lax.rem(g + 2, 3))

        return carry

    lax.fori_loop(0, FULL_BATCHES, body, 0)
    scat_wait()
    off_r = base + FULL_BATCHES * BATCH
    pltpu.sync_copy(src2_ref.at[pl.ds(src_base + FULL_BATCHES * BATCH, REM)], nv_r)
    pltpu.sync_copy(dst_ref.at[pl.ds(off_r, REM)], ev_r)
    pltpu.async_copy(h_ref.at[nv_r], rows_r, sem_g).wait()
    pltpu.sync_copy(rows_r, acc.at[ev_r], add=True)
    plsc.subcore_barrier()
    pltpu.sync_copy(acc.at[pl.ds(r0, ROWS_PER_SUB)],
                    out_ref.at[c, pl.ds(r0, ROWS_PER_SUB)])


def _sc_segsum(h_flat, src2, dst, zeros128):
    return pl.kernel(
        _segsum_body,
        out_type=jax.ShapeDtypeStruct((NC, N_NODES, 128), _f32),
        mesh=_sc_mesh(),
        compiler_params=pltpu.CompilerParams(use_tc_tiling_on_sc=False),
        scratch_types=[
            pltpu.VMEM((3, BATCH), jnp.int32),
            pltpu.VMEM((3, BATCH), jnp.int32),
            pltpu.VMEM((2, BATCH, 128), _f32),
            pltpu.VMEM((REM,), jnp.int32),
            pltpu.VMEM((REM,), jnp.int32),
            pltpu.VMEM((REM, 128), _f32),
            pltpu.VMEM_SHARED((N_NODES, 128), _f32),
            pltpu.SemaphoreType.DMA,
            pltpu.SemaphoreType.DMA,
            pltpu.SemaphoreType.DMA,
        ],
    )(h_flat, src2, dst, zeros128)


BM = 1000
GRID = N_NODES // BM


def _chunked_spec():
    return pl.BlockSpec((NC, BM, 128), lambda i: (0, i, 0))


def _w_spec(k):
    return pl.BlockSpec((k, 256), lambda i: (0, 0))


def _b_spec():
    return pl.BlockSpec((1, 256), lambda i: (0, 0))


def _hist_spec():
    return pl.BlockSpec((BM, 16), lambda i: (i, 0))


def _a_spec():
    return pl.BlockSpec((1, 1), lambda i: (0, 0), memory_space=pltpu.SMEM)


def _write_chunked(out_ref, v):
    out_ref[0] = v[:, :128]
    out_ref[1] = v[:, 128:]


def _cat(ref):
    return jnp.concatenate([ref[0], ref[1]], axis=1)


def _tc_a1_body(x_ref, w_ref, b_ref, out_ref):
    h = jnp.dot(x_ref[...], w_ref[...], preferred_element_type=_f32) + b_ref[...]
    _write_chunked(out_ref, h)


def _tc_a1(x, W, b2d):
    return pl.pallas_call(
        _tc_a1_body,
        grid=(GRID,),
        in_specs=[pl.BlockSpec((BM, 128), lambda i: (i, 0)), _w_spec(128), _b_spec()],
        out_specs=_chunked_spec(),
        out_shape=jax.ShapeDtypeStruct((NC, N_NODES, 128), _f32),
    )(x, W, b2d)


def _tc_b_body(emit_e, se_ref, h_ref, hist_ref, w_ref, b_ref, a_ref, *out_refs):
    a = a_ref[0, 0]
    hist = hist_ref[:, 0:1]
    de_inv = jnp.where(hist > 0, 1.0 / hist, 0.0)
    e_head = _prelu(de_inv * _cat(se_ref), a)
    e_tail = _prelu(_cat(h_ref), a)
    h2h = jnp.dot(e_head, w_ref[...], preferred_element_type=_f32) + b_ref[...]
    h2t = jnp.dot(e_tail, w_ref[...], preferred_element_type=_f32) + b_ref[...]
    _write_chunked(out_refs[0], h2h)
    _write_chunked(out_refs[1], h2t)
    if emit_e:
        out_refs[2][...] = e_head


def _tc_b(se, h, hist_e, W, b2d, a2d, emit_e):
    out_shapes = [jax.ShapeDtypeStruct((NC, N_NODES, 128), _f32),
                  jax.ShapeDtypeStruct((NC, N_NODES, 128), _f32)]
    out_specs = [_chunked_spec(), _chunked_spec()]
    if emit_e:
        out_shapes.append(jax.ShapeDtypeStruct((N_NODES, 256), _f32))
        out_specs.append(pl.BlockSpec((BM, 256), lambda i: (i, 0)))
    return pl.pallas_call(
        functools.partial(_tc_b_body, emit_e),
        grid=(GRID,),
        in_specs=[_chunked_spec(), _chunked_spec(), _hist_spec(),
                  _w_spec(256), _b_spec(), _a_spec()],
        out_specs=out_specs,
        out_shape=out_shapes,
    )(se, h, hist_e, W, b2d, a2d)


def _tc_c1_body(sn_ref, h2t_ref, hist_ref, w_ref, b_ref, a_ref, out_ref):
    a = a_ref[0, 0]
    dn_inv = 1.0 / (hist_ref[:, 0:1] + 1.0)
    n1 = _prelu(dn_inv * (_cat(sn_ref) + _cat(h2t_ref)), a)
    h = jnp.dot(n1, w_ref[...], preferred_element_type=_f32) + b_ref[...]
    _write_chunked(out_ref, h)


def _tc_c1(sn, h2t, hist_n, W, b2d, a2d):
    return pl.pallas_call(
        _tc_c1_body,
        grid=(GRID,),
        in_specs=[_chunked_spec(), _chunked_spec(), _hist_spec(),
                  _w_spec(256), _b_spec(), _a_spec()],
        out_specs=_chunked_spec(),
        out_shape=jax.ShapeDtypeStruct((NC, N_NODES, 128), _f32),
    )(sn, h2t, hist_n, W, b2d, a2d)


def _tc_c2_body(sn_ref, h2t_ref, hist_ref, a_ref, out_ref):
    a = a_ref[0, 0]
    dn_inv = 1.0 / (hist_ref[:, 0:1] + 1.0)
    out_ref[...] = _prelu(dn_inv * (_cat(sn_ref) + _cat(h2t_ref)), a)


def _tc_c2(sn, h2t, hist_n, a2d):
    return pl.pallas_call(
        _tc_c2_body,
        grid=(GRID,),
        in_specs=[_chunked_spec(), _chunked_spec(), _hist_spec(), _a_spec()],
        out_specs=pl.BlockSpec((BM, 256), lambda i: (i, 0)),
        out_shape=jax.ShapeDtypeStruct((N_NODES, 256), _f32),
    )(sn, h2t, hist_n, a2d)


def kernel(x, W1_n2e, b1_n2e, W1_e2n, b1_e2n, W2_n2e, b2_n2e, W2_e2n, b2_e2n,
           prelu_a, hyperedge_index, num_nodes, num_edges):
    del num_nodes, num_edges
    ni = hyperedge_index[0]
    ei = hyperedge_index[1]
    ni2 = jnp.concatenate([ni, ni + N_NODES])
    ei2 = jnp.concatenate([ei, ei + N_NODES])

    ones16 = jnp.ones((BATCH, 16), _f32)
    zeros16 = jnp.zeros((ROWS_PER_SUB, 16), _f32)
    zeros128 = jnp.zeros((ROWS_PER_SUB, 128), _f32)
    a2d = prelu_a.reshape(1, 1)

    hists = _sc_hist(hyperedge_index.reshape(2 * E_PAIRS), ones16, zeros16)
    hist_n = hists[0]
    hist_e = hists[1]

    h1 = _tc_a1(x, W1_n2e, b1_n2e.reshape(1, 256))
    s_e1 = _sc_segsum(h1.reshape(NC * N_NODES, 128), ni2, ei, zeros128)
    h2h1, h2t1 = _tc_b(s_e1, h1, hist_e, W1_e2n, b1_e2n.reshape(1, 256), a2d,
                       emit_e=False)
    s_n1 = _sc_segsum(h2h1.reshape(NC * N_NODES, 128), ei2, ni, zeros128)
    hA2 = _tc_c1(s_n1, h2t1, hist_n, W2_n2e, b2_n2e.reshape(1, 256), a2d)
    s_e2 = _sc_segsum(hA2.reshape(NC * N_NODES, 128), ni2, ei, zeros128)
    h2h2, h2t2, e_out = _tc_b(s_e2, hA2, hist_e, W2_e2n, b2_e2n.reshape(1, 256),
                              a2d, emit_e=True)
    s_n2 = _sc_segsum(h2h2.reshape(NC * N_NODES, 128), ei2, ni, zeros128)
    n_out = _tc_c2(s_n2, h2t2, hist_n, a2d)
    return (n_out, e_out)


# speedup vs baseline: 15.6395x; 1.0077x over previous
import functools

import jax
import jax.numpy as jnp
from jax import lax
from jax.experimental import pallas as pl
from jax.experimental.pallas import tpu as pltpu
from jax.experimental.pallas import tpu_sc as plsc

N_NODES = 10000
N_EDGES = 10000
E_PAIRS = 320000
NC = 2
NS = 16
PER_SUB = E_PAIRS // NS
BATCH = 128
FULL_BATCHES = PER_SUB // BATCH
REM = PER_SUB - FULL_BATCHES * BATCH
ROWS_PER_SUB = N_NODES // NS
GROUP = 12
GROUPS = 13

_f32 = jnp.float32


def _prelu(v, a):
    return jnp.where(v >= 0, v, a * v)


def _sc_mesh():
    return plsc.VectorSubcoreMesh(core_axis_name="c", subcore_axis_name="s")


def _hist_body(hei_ref, ones_ref, zeros_ref, out_ref,
               idx2, idx_r, ones_v, ones_r, acc, sem_i):
    c = lax.axis_index("c")
    s = lax.axis_index("s")
    r0 = s * ROWS_PER_SUB
    pltpu.sync_copy(zeros_ref, acc.at[pl.ds(r0, ROWS_PER_SUB)])
    pltpu.sync_copy(ones_ref, ones_v)
    pltpu.sync_copy(ones_ref.at[pl.ds(0, REM)], ones_r)
    plsc.subcore_barrier()
    base = c * E_PAIRS + s * PER_SUB

    def idx_start(g, slot):
        pltpu.async_copy(hei_ref.at[pl.ds(base + g * BATCH, BATCH)],
                         idx2.at[slot], sem_i)

    def idx_wait(slot):
        pltpu.make_async_copy(hei_ref.at[pl.ds(0, BATCH)], idx2.at[slot],
                              sem_i).wait()

    idx_start(0, 0)
    idx_start(1, 1)

    def body(g, carry):
        p = lax.rem(g, 2)
        idx_wait(p)
        pltpu.sync_copy(ones_v, acc.at[idx2.at[p]], add=True)

        @pl.when(g + 2 < FULL_BATCHES)
        def _():
            idx_start(g + 2, p)

        return carry

    lax.fori_loop(0, FULL_BATCHES, body, 0)
    pltpu.sync_copy(hei_ref.at[pl.ds(base + FULL_BATCHES * BATCH, REM)], idx_r)
    pltpu.sync_copy(ones_r, acc.at[idx_r], add=True)
    plsc.subcore_barrier()
    pltpu.sync_copy(acc.at[pl.ds(r0, ROWS_PER_SUB)],
                    out_ref.at[c, pl.ds(r0, ROWS_PER_SUB)])


def _sc_hist(hei, ones16, zeros16):
    return pl.kernel(
        _hist_body,
        out_type=jax.ShapeDtypeStruct((NC, N_NODES, 16), _f32),
        mesh=_sc_mesh(),
        compiler_params=pltpu.CompilerParams(use_tc_tiling_on_sc=False),
        scratch_types=[
            pltpu.VMEM((2, BATCH), jnp.int32),
            pltpu.VMEM((REM,), jnp.int32),
            pltpu.VMEM((BATCH, 16), _f32),
            pltpu.VMEM((REM, 16), _f32),
            pltpu.VMEM_SHARED((N_NODES, 16), _f32),
            pltpu.SemaphoreType.DMA,
        ],
    )(hei, ones16, zeros16)


def _segsum_body(h_ref, inter_ref, zeros_ref, out_ref,
                 ibuf, rows2, acc, sem_g, sem_i, sem_s):
    c = lax.axis_index("c")
    s = lax.axis_index("s")
    r0 = s * ROWS_PER_SUB
    pltpu.sync_copy(zeros_ref, acc.at[pl.ds(r0, ROWS_PER_SUB)])
    plsc.subcore_barrier()

    b_lo = (2500 * s) // 16
    b_hi = (2500 * (s + 1)) // 16
    leftover = b_hi - b_lo - GROUPS * GROUP
    row0 = c * 2500 + b_lo

    def idx_start(grp, slot):
        pltpu.async_copy(inter_ref.at[pl.ds(row0 + grp * GROUP, GROUP)],
                         ibuf.at[slot], sem_i)

    def idx_wait(slot):
        pltpu.make_async_copy(inter_ref.at[pl.ds(0, GROUP)], ibuf.at[slot],
                              sem_i).wait()

    def gath_start(slot, j, p):
        pltpu.async_copy(h_ref.at[ibuf.at[slot, j, 0]], rows2.at[p], sem_g)

    def gath_wait(p):
        pltpu.make_async_copy(h_ref.at[ibuf.at[0, 0, 0]], rows2.at[p],
                              sem_g).wait()

    def scat_wait():
        pltpu.make_async_copy(rows2.at[0], acc.at[ibuf.at[0, 0, 1]],
                              sem_s).wait()

    pltpu.sync_copy(inter_ref.at[pl.ds(row0, GROUP)], ibuf.at[0])
    gath_start(0, 0, 0)

    def group(grp, carry):
        slot = lax.rem(grp, 2)
        nslot = 1 - slot
        for j in range(GROUP):
            m = grp * GROUP + j
            pj = j % 2
            gath_wait(pj)

            @pl.when(m >= 1)
            def _():
                scat_wait()

            if j == 0:
                @pl.when(grp + 1 < GROUPS)
                def _():
                    idx_start(grp + 1, nslot)

            if j + 1 < GROUP:
                gath_start(slot, j + 1, 1 - pj)
            else:
                @pl.when(grp + 1 < GROUPS)
                def _():
                    idx_wait(nslot)
                    gath_start(nslot, 0, 1 - pj)

            pltpu.async_copy(rows2.at[pj], acc.at[ibuf.at[slot, j, 1]], sem_s,
                             add=True)
        return carry

    lax.fori_loop(0, GROUPS, group, 0)
    scat_wait()

    @pl.when(leftover > 0)
    def _():
        pltpu.sync_copy(inter_ref.at[pl.ds(row0 + GROUPS * GROUP, 1)],
                        ibuf.at[0, pl.ds(0, 1)])
        pltpu.async_copy(h_ref.at[ibuf.at[0, 0, 0]], rows2.at[0], sem_g).wait()
        pltpu.sync_copy(rows2.at[0], acc.at[ibuf.at[0, 0, 1]], add=True)

    plsc.subcore_barrier()
    pltpu.sync_copy(acc.at[pl.ds(r0, ROWS_PER_SUB)],
                    out_ref.at[c, pl.ds(r0, ROWS_PER_SUB)])


def _sc_segsum(h_flat, inter, zeros128):
    return pl.kernel(
        _segsum_body,
        out_type=jax.ShapeDtypeStruct((NC, N_NODES, 128), _f32),
        mesh=_sc_mesh(),
        compiler_params=pltpu.CompilerParams(use_tc_tiling_on_sc=False),
        scratch_types=[
            pltpu.VMEM((2, GROUP, 2, BATCH), jnp.int32),
            pltpu.VMEM((2, BATCH, 128), _f32),
            pltpu.VMEM_SHARED((N_NODES, 128), _f32),
            pltpu.SemaphoreType.DMA,
            pltpu.SemaphoreType.DMA,
            pltpu.SemaphoreType.DMA,
        ],
    )(h_flat, inter, zeros128)


BM = 1000
GRID = N_NODES // BM


def _chunked_spec():
    return pl.BlockSpec((NC, BM, 128), lambda i: (0, i, 0))


def _w_spec(k):
    return pl.BlockSpec((k, 256), lambda i: (0, 0))


def _b_spec():
    return pl.BlockSpec((1, 256), lambda i: (0, 0))


def _hist_spec():
    return pl.BlockSpec((BM, 16), lambda i: (i, 0))


def _a_spec():
    return pl.BlockSpec((1, 1), lambda i: (0, 0), memory_space=pltpu.SMEM)


def _write_chunked(out_ref, v):
    out_ref[0] = v[:, :128]
    out_ref[1] = v[:, 128:]


def _cat(ref):
    return jnp.concatenate([ref[0], ref[1]], axis=1)


def _tc_a1_body(x_ref, w_ref, b_ref, out_ref):
    h = jnp.dot(x_ref[...], w_ref[...], preferred_element_type=_f32) + b_ref[...]
    _write_chunked(out_ref, h)


def _tc_a1(x, W, b2d):
    return pl.pallas_call(
        _tc_a1_body,
        grid=(GRID,),
        in_specs=[pl.BlockSpec((BM, 128), lambda i: (i, 0)), _w_spec(128), _b_spec()],
        out_specs=_chunked_spec(),
        out_shape=jax.ShapeDtypeStruct((NC, N_NODES, 128), _f32),
    )(x, W, b2d)


def _tc_b_body(emit_e, se_ref, h_ref, hist_ref, w_ref, b_ref, a_ref, *out_refs):
    a = a_ref[0, 0]
    hist = hist_ref[:, 0:1]
    de_inv = jnp.where(hist > 0, 1.0 / hist, 0.0)
    e_head = _prelu(de_inv * _cat(se_ref), a)
    e_tail = _prelu(_cat(h_ref), a)
    h2h = jnp.dot(e_head, w_ref[...], preferred_element_type=_f32) + b_ref[...]
    h2t = jnp.dot(e_tail, w_ref[...], preferred_element_type=_f32) + b_ref[...]
    _write_chunked(out_refs[0], h2h)
    _write_chunked(out_refs[1], h2t)
    if emit_e:
        out_refs[2][...] = e_head


def _tc_b(se, h, hist_e, W, b2d, a2d, emit_e):
    out_shapes = [jax.ShapeDtypeStruct((NC, N_NODES, 128), _f32),
                  jax.ShapeDtypeStruct((NC, N_NODES, 128), _f32)]
    out_specs = [_chunked_spec(), _chunked_spec()]
    if emit_e:
        out_shapes.append(jax.ShapeDtypeStruct((N_NODES, 256), _f32))
        out_specs.append(pl.BlockSpec((BM, 256), lambda i: (i, 0)))
    return pl.pallas_call(
        functools.partial(_tc_b_body, emit_e),
        grid=(GRID,),
        in_specs=[_chunked_spec(), _chunked_spec(), _hist_spec(),
                  _w_spec(256), _b_spec(), _a_spec()],
        out_specs=out_specs,
        out_shape=out_shapes,
    )(se, h, hist_e, W, b2d, a2d)


def _tc_c1_body(sn_ref, h2t_ref, hist_ref, w_ref, b_ref, a_ref, out_ref):
    a = a_ref[0, 0]
    dn_inv = 1.0 / (hist_ref[:, 0:1] + 1.0)
    n1 = _prelu(dn_inv * (_cat(sn_ref) + _cat(h2t_ref)), a)
    h = jnp.dot(n1, w_ref[...], preferred_element_type=_f32) + b_ref[...]
    _write_chunked(out_ref, h)


def _tc_c1(sn, h2t, hist_n, W, b2d, a2d):
    return pl.pallas_call(
        _tc_c1_body,
        grid=(GRID,),
        in_specs=[_chunked_spec(), _chunked_spec(), _hist_spec(),
                  _w_spec(256), _b_spec(), _a_spec()],
        out_specs=_chunked_spec(),
        out_shape=jax.ShapeDtypeStruct((NC, N_NODES, 128), _f32),
    )(sn, h2t, hist_n, W, b2d, a2d)


def _tc_c2_body(sn_ref, h2t_ref, hist_ref, a_ref, out_ref):
    a = a_ref[0, 0]
    dn_inv = 1.0 / (hist_ref[:, 0:1] + 1.0)
    out_ref[...] = _prelu(dn_inv * (_cat(sn_ref) + _cat(h2t_ref)), a)


def _tc_c2(sn, h2t, hist_n, a2d):
    return pl.pallas_call(
        _tc_c2_body,
        grid=(GRID,),
        in_specs=[_chunked_spec(), _chunked_spec(), _hist_spec(), _a_spec()],
        out_specs=pl.BlockSpec((BM, 256), lambda i: (i, 0)),
        out_shape=jax.ShapeDtypeStruct((N_NODES, 256), _f32),
    )(sn, h2t, hist_n, a2d)


def kernel(x, W1_n2e, b1_n2e, W1_e2n, b1_e2n, W2_n2e, b2_n2e, W2_e2n, b2_e2n,
           prelu_a, hyperedge_index, num_nodes, num_edges):
    del num_nodes, num_edges
    ni = hyperedge_index[0]
    ei = hyperedge_index[1]

    def make_inter(src, dst):
        sb = src.reshape(E_PAIRS // BATCH, BATCH)
        db = dst.reshape(E_PAIRS // BATCH, BATCH)
        return jnp.concatenate(
            [jnp.stack([sb, db], axis=1),
             jnp.stack([sb + N_NODES, db], axis=1)], axis=0)

    inter_n2e = make_inter(ni, ei)
    inter_e2n = make_inter(ei, ni)

    ones16 = jnp.ones((BATCH, 16), _f32)
    zeros16 = jnp.zeros((ROWS_PER_SUB, 16), _f32)
    zeros128 = jnp.zeros((ROWS_PER_SUB, 128), _f32)
    a2d = prelu_a.reshape(1, 1)

    hists = _sc_hist(hyperedge_index.reshape(2 * E_PAIRS), ones16, zeros16)
    hist_n = hists[0]
    hist_e = hists[1]

    h1 = _tc_a1(x, W1_n2e, b1_n2e.reshape(1, 256))
    s_e1 = _sc_segsum(h1.reshape(NC * N_NODES, 128), inter_n2e, zeros128)
    h2h1, h2t1 = _tc_b(s_e1, h1, hist_e, W1_e2n, b1_e2n.reshape(1, 256), a2d,
                       emit_e=False)
    s_n1 = _sc_segsum(h2h1.reshape(NC * N_NODES, 128), inter_e2n, zeros128)
    hA2 = _tc_c1(s_n1, h2t1, hist_n, W2_n2e, b2_n2e.reshape(1, 256), a2d)
    s_e2 = _sc_segsum(hA2.reshape(NC * N_NODES, 128), inter_n2e, zeros128)
    h2h2, h2t2, e_out = _tc_b(s_e2, hA2, hist_e, W2_e2n, b2_e2n.reshape(1, 256),
                              a2d, emit_e=True)
    s_n2 = _sc_segsum(h2h2.reshape(NC * N_NODES, 128), inter_e2n, zeros128)
    n_out = _tc_c2(s_n2, h2t2, hist_n, a2d)
    return (n_out, e_out)


# speedup vs baseline: 17.4269x; 1.1143x over previous
import functools

import jax
import jax.numpy as jnp
from jax import lax
from jax.experimental import pallas as pl
from jax.experimental.pallas import tpu as pltpu
from jax.experimental.pallas import tpu_sc as plsc

N_NODES = 10000
N_EDGES = 10000
E_PAIRS = 320000
NC = 2
NS = 16
PER_SUB = E_PAIRS // NS
BATCH = 128
FULL_BATCHES = PER_SUB // BATCH
REM = PER_SUB - FULL_BATCHES * BATCH
ROWS_PER_SUB = N_NODES // NS
GROUP = 12
GROUPS = 13
GROUP_P = 6
GROUPS_P = 13

_f32 = jnp.float32


def _prelu(v, a):
    return jnp.where(v >= 0, v, a * v)


def _sc_mesh():
    return plsc.VectorSubcoreMesh(core_axis_name="c", subcore_axis_name="s")


def _hist_body(hei_ref, ones_ref, zeros_ref, out_ref,
               idx2, idx_r, ones_v, ones_r, acc, sem_i):
    c = lax.axis_index("c")
    s = lax.axis_index("s")
    r0 = s * ROWS_PER_SUB
    pltpu.sync_copy(zeros_ref, acc.at[pl.ds(r0, ROWS_PER_SUB)])
    pltpu.sync_copy(ones_ref, ones_v)
    pltpu.sync_copy(ones_ref.at[pl.ds(0, REM)], ones_r)
    plsc.subcore_barrier()
    base = c * E_PAIRS + s * PER_SUB

    def idx_start(g, slot):
        pltpu.async_copy(hei_ref.at[pl.ds(base + g * BATCH, BATCH)],
                         idx2.at[slot], sem_i)

    def idx_wait(slot):
        pltpu.make_async_copy(hei_ref.at[pl.ds(0, BATCH)], idx2.at[slot],
                              sem_i).wait()

    idx_start(0, 0)
    idx_start(1, 1)

    def body(g, carry):
        p = lax.rem(g, 2)
        idx_wait(p)
        pltpu.sync_copy(ones_v, acc.at[idx2.at[p]], add=True)

        @pl.when(g + 2 < FULL_BATCHES)
        def _():
            idx_start(g + 2, p)

        return carry

    lax.fori_loop(0, FULL_BATCHES, body, 0)
    pltpu.sync_copy(hei_ref.at[pl.ds(base + FULL_BATCHES * BATCH, REM)], idx_r)
    pltpu.sync_copy(ones_r, acc.at[idx_r], add=True)
    plsc.subcore_barrier()
    pltpu.sync_copy(acc.at[pl.ds(r0, ROWS_PER_SUB)],
                    out_ref.at[c, pl.ds(r0, ROWS_PER_SUB)])


def _sc_hist(hei, ones16, zeros16):
    return pl.kernel(
        _hist_body,
        out_type=jax.ShapeDtypeStruct((NC, N_NODES, 16), _f32),
        mesh=_sc_mesh(),
        compiler_params=pltpu.CompilerParams(use_tc_tiling_on_sc=False),
        scratch_types=[
            pltpu.VMEM((2, BATCH), jnp.int32),
            pltpu.VMEM((REM,), jnp.int32),
            pltpu.VMEM((BATCH, 16), _f32),
            pltpu.VMEM((REM, 16), _f32),
            pltpu.VMEM_SHARED((N_NODES, 16), _f32),
            pltpu.SemaphoreType.DMA,
        ],
    )(hei, ones16, zeros16)


def _make_segsum_body(bpc, group, groups):
  def _segsum_body(h_ref, inter_ref, zeros_ref, out_ref,
                   ibuf, rows2, acc, sem_g, sem_i, sem_s):
      c = lax.axis_index("c")
      s = lax.axis_index("s")
      r0 = s * ROWS_PER_SUB
      pltpu.sync_copy(zeros_ref, acc.at[pl.ds(r0, ROWS_PER_SUB)])
      plsc.subcore_barrier()

      b_lo = (bpc * s) // 16
      b_hi = (bpc * (s + 1)) // 16
      leftover = b_hi - b_lo - groups * group
      row0 = c * bpc + b_lo

      def idx_start(grp, slot):
          pltpu.async_copy(inter_ref.at[pl.ds(row0 + grp * group, group)],
                           ibuf.at[slot], sem_i)

      def idx_wait(slot):
          pltpu.make_async_copy(inter_ref.at[pl.ds(0, group)], ibuf.at[slot],
                                sem_i).wait()

      def gath_start(slot, j, p):
          pltpu.async_copy(h_ref.at[ibuf.at[slot, j, 0]], rows2.at[p], sem_g)

      def gath_wait(p):
          pltpu.make_async_copy(h_ref.at[ibuf.at[0, 0, 0]], rows2.at[p],
                                sem_g).wait()

      def scat_wait():
          pltpu.make_async_copy(rows2.at[0], acc.at[ibuf.at[0, 0, 1]],
                                sem_s).wait()

      pltpu.sync_copy(inter_ref.at[pl.ds(row0, group)], ibuf.at[0])
      gath_start(0, 0, 0)

      def group_fn(grp, carry):
          slot = lax.rem(grp, 2)
          nslot = 1 - slot
          for j in range(group):
              m = grp * group + j
              pj = j % 2
              gath_wait(pj)

              @pl.when(m >= 1)
              def _():
                  scat_wait()

              if j == 0:
                  @pl.when(grp + 1 < groups)
                  def _():
                      idx_start(grp + 1, nslot)

              if j + 1 < group:
                  gath_start(slot, j + 1, 1 - pj)
              else:
                  @pl.when(grp + 1 < groups)
                  def _():
                      idx_wait(nslot)
                      gath_start(nslot, 0, 1 - pj)

              pltpu.async_copy(rows2.at[pj], acc.at[ibuf.at[slot, j, 1]], sem_s,
                               add=True)
          return carry

      lax.fori_loop(0, groups, group_fn, 0)
      scat_wait()

      @pl.when(leftover > 0)
      def _():
          pltpu.sync_copy(inter_ref.at[pl.ds(row0 + groups * group, 1)],
                          ibuf.at[0, pl.ds(0, 1)])
          pltpu.async_copy(h_ref.at[ibuf.at[0, 0, 0]], rows2.at[0], sem_g).wait()
          pltpu.sync_copy(rows2.at[0], acc.at[ibuf.at[0, 0, 1]], add=True)

      plsc.subcore_barrier()
      pltpu.sync_copy(acc.at[pl.ds(r0, ROWS_PER_SUB)],
                      out_ref.at[c, pl.ds(r0, ROWS_PER_SUB)])


  return _segsum_body


def _sc_segsum(h_flat, inter, zeros128):
    return pl.kernel(
        _make_segsum_body(2500, GROUP, GROUPS),
        out_type=jax.ShapeDtypeStruct((NC, N_NODES, 128), _f32),
        mesh=_sc_mesh(),
        compiler_params=pltpu.CompilerParams(use_tc_tiling_on_sc=False),
        scratch_types=[
            pltpu.VMEM((2, GROUP, 2, BATCH), jnp.int32),
            pltpu.VMEM((2, BATCH, 128), _f32),
            pltpu.VMEM_SHARED((N_NODES, 128), _f32),
            pltpu.SemaphoreType.DMA,
            pltpu.SemaphoreType.DMA,
            pltpu.SemaphoreType.DMA,
        ],
    )(h_flat, inter, zeros128)


def _sc_segsum_pairs(table, inter, zeros128):
    return pl.kernel(
        _make_segsum_body(1250, GROUP_P, GROUPS_P),
        out_type=jax.ShapeDtypeStruct((NC, N_NODES, 128), _f32),
        mesh=_sc_mesh(),
        compiler_params=pltpu.CompilerParams(use_tc_tiling_on_sc=False),
        scratch_types=[
            pltpu.VMEM((2, GROUP_P, 2, BATCH), jnp.int32),
            pltpu.VMEM((2, BATCH, 128), _f32),
            pltpu.VMEM_SHARED((N_NODES, 128), _f32),
            pltpu.SemaphoreType.DMA,
            pltpu.SemaphoreType.DMA,
            pltpu.SemaphoreType.DMA,
        ],
    )(table, inter, zeros128)


BM = 1000
GRID = N_NODES // BM


def _chunked_spec():
    return pl.BlockSpec((NC, BM, 128), lambda i: (0, i, 0))


def _w_spec(k):
    return pl.BlockSpec((k, 256), lambda i: (0, 0))


def _b_spec():
    return pl.BlockSpec((1, 256), lambda i: (0, 0))


def _hist_spec():
    return pl.BlockSpec((BM, 16), lambda i: (i, 0))


def _a_spec():
    return pl.BlockSpec((1, 1), lambda i: (0, 0), memory_space=pltpu.SMEM)


def _write_chunked(out_ref, v):
    out_ref[0] = v[:, :128]
    out_ref[1] = v[:, 128:]


def _cat(ref):
    return jnp.concatenate([ref[0], ref[1]], axis=1)


def _tc_a1_body(x_ref, w_ref, b_ref, out_ref):
    h = jnp.dot(x_ref[...], w_ref[...], preferred_element_type=_f32) + b_ref[...]
    _write_chunked(out_ref, h)


def _tc_a1(x, W, b2d):
    return pl.pallas_call(
        _tc_a1_body,
        grid=(GRID,),
        in_specs=[pl.BlockSpec((BM, 128), lambda i: (i, 0)), _w_spec(128), _b_spec()],
        out_specs=_chunked_spec(),
        out_shape=jax.ShapeDtypeStruct((NC, N_NODES, 128), _f32),
    )(x, W, b2d)


def _tc_b1_body(sx_ref, h_ref, hist_ref, wa_ref, ba_ref, w_ref, b_ref, a_ref,
                h2h_ref, h2t_ref):
    a = a_ref[0, 0]
    hist = hist_ref[:, 0:1]
    de_inv = jnp.where(hist > 0, 1.0 / hist, 0.0)
    sx = sx_ref[0] + sx_ref[1]
    esum = jnp.dot(sx, wa_ref[...], preferred_element_type=_f32) + hist * ba_ref[...]
    e_head = _prelu(de_inv * esum, a)
    e_tail = _prelu(_cat(h_ref), a)
    h2h = jnp.dot(e_head, w_ref[...], preferred_element_type=_f32) + b_ref[...]
    h2t = jnp.dot(e_tail, w_ref[...], preferred_element_type=_f32) + b_ref[...]
    _write_chunked(h2h_ref, h2h)
    _write_chunked(h2t_ref, h2t)


def _tc_b1(sx, h, hist_e, Wa, ba2d, W, b2d, a2d):
    return pl.pallas_call(
        _tc_b1_body,
        grid=(GRID,),
        in_specs=[_chunked_spec(), _chunked_spec(), _hist_spec(),
                  _w_spec(128), _b_spec(), _w_spec(256), _b_spec(), _a_spec()],
        out_specs=[_chunked_spec(), _chunked_spec()],
        out_shape=[jax.ShapeDtypeStruct((NC, N_NODES, 128), _f32),
                   jax.ShapeDtypeStruct((NC, N_NODES, 128), _f32)],
    )(sx, h, hist_e, Wa, ba2d, W, b2d, a2d)


def _tc_b_body(emit_e, se_ref, h_ref, hist_ref, w_ref, b_ref, a_ref, *out_refs):
    a = a_ref[0, 0]
    hist = hist_ref[:, 0:1]
    de_inv = jnp.where(hist > 0, 1.0 / hist, 0.0)
    e_head = _prelu(de_inv * _cat(se_ref), a)
    e_tail = _prelu(_cat(h_ref), a)
    h2h = jnp.dot(e_head, w_ref[...], preferred_element_type=_f32) + b_ref[...]
    h2t = jnp.dot(e_tail, w_ref[...], preferred_element_type=_f32) + b_ref[...]
    _write_chunked(out_refs[0], h2h)
    _write_chunked(out_refs[1], h2t)
    if emit_e:
        out_refs[2][...] = e_head


def _tc_b(se, h, hist_e, W, b2d, a2d, emit_e):
    out_shapes = [jax.ShapeDtypeStruct((NC, N_NODES, 128), _f32),
                  jax.ShapeDtypeStruct((NC, N_NODES, 128), _f32)]
    out_specs = [_chunked_spec(), _chunked_spec()]
    if emit_e:
        out_shapes.append(jax.ShapeDtypeStruct((N_NODES, 256), _f32))
        out_specs.append(pl.BlockSpec((BM, 256), lambda i: (i, 0)))
    return pl.pallas_call(
        functools.partial(_tc_b_body, emit_e),
        grid=(GRID,),
        in_specs=[_chunked_spec(), _chunked_spec(), _hist_spec(),
                  _w_spec(256), _b_spec(), _a_spec()],
        out_specs=out_specs,
        out_shape=out_shapes,
    )(se, h, hist_e, W, b2d, a2d)


def _tc_c1_body(sn_ref, h2t_ref, hist_ref, w_ref, b_ref, a_ref, out_ref):
    a = a_ref[0, 0]
    dn_inv = 1.0 / (hist_ref[:, 0:1] + 1.0)
    n1 = _prelu(dn_inv * (_cat(sn_ref) + _cat(h2t_ref)), a)
    h = jnp.dot(n1, w_ref[...], preferred_element_type=_f32) + b_ref[...]
    _write_chunked(out_ref, h)


def _tc_c1(sn, h2t, hist_n, W, b2d, a2d):
    return pl.pallas_call(
        _tc_c1_body,
        grid=(GRID,),
        in_specs=[_chunked_spec(), _chunked_spec(), _hist_spec(),
                  _w_spec(256), _b_spec(), _a_spec()],
        out_specs=_chunked_spec(),
        out_shape=jax.ShapeDtypeStruct((NC, N_NODES, 128), _f32),
    )(sn, h2t, hist_n, W, b2d, a2d)


def _tc_c2_body(sn_ref, h2t_ref, hist_ref, a_ref, out_ref):
    a = a_ref[0, 0]
    dn_inv = 1.0 / (hist_ref[:, 0:1] + 1.0)
    out_ref[...] = _prelu(dn_inv * (_cat(sn_ref) + _cat(h2t_ref)), a)


def _tc_c2(sn, h2t, hist_n, a2d):
    return pl.pallas_call(
        _tc_c2_body,
        grid=(GRID,),
        in_specs=[_chunked_spec(), _chunked_spec(), _hist_spec(), _a_spec()],
        out_specs=pl.BlockSpec((BM, 256), lambda i: (i, 0)),
        out_shape=jax.ShapeDtypeStruct((N_NODES, 256), _f32),
    )(sn, h2t, hist_n, a2d)


def kernel(x, W1_n2e, b1_n2e, W1_e2n, b1_e2n, W2_n2e, b2_n2e, W2_e2n, b2_e2n,
           prelu_a, hyperedge_index, num_nodes, num_edges):
    del num_nodes, num_edges
    ni = hyperedge_index[0]
    ei = hyperedge_index[1]

    def make_inter(src, dst):
        sb = src.reshape(E_PAIRS // BATCH, BATCH)
        db = dst.reshape(E_PAIRS // BATCH, BATCH)
        return jnp.concatenate(
            [jnp.stack([sb, db], axis=1),
             jnp.stack([sb + N_NODES, db], axis=1)], axis=0)

    inter_n2e = make_inter(ni, ei)
    inter_e2n = make_inter(ei, ni)

    ones16 = jnp.ones((BATCH, 16), _f32)
    zeros16 = jnp.zeros((ROWS_PER_SUB, 16), _f32)
    zeros128 = jnp.zeros((ROWS_PER_SUB, 128), _f32)
    a2d = prelu_a.reshape(1, 1)

    hists = _sc_hist(hyperedge_index.reshape(2 * E_PAIRS), ones16, zeros16)
    hist_n = hists[0]
    hist_e = hists[1]

    h1 = _tc_a1(x, W1_n2e, b1_n2e.reshape(1, 256))
    s_x1 = _sc_segsum_pairs(x, inter_n2e, zeros128)
    h2h1, h2t1 = _tc_b1(s_x1, h1, hist_e, W1_n2e, b1_n2e.reshape(1, 256),
                        W1_e2n, b1_e2n.reshape(1, 256), a2d)
    s_n1 = _sc_segsum(h2h1.reshape(NC * N_NODES, 128), inter_e2n, zeros128)
    hA2 = _tc_c1(s_n1, h2t1, hist_n, W2_n2e, b2_n2e.reshape(1, 256), a2d)
    s_e2 = _sc_segsum(hA2.reshape(NC * N_NODES, 128), inter_n2e, zeros128)
    h2h2, h2t2, e_out = _tc_b(s_e2, hA2, hist_e, W2_e2n, b2_e2n.reshape(1, 256),
                              a2d, emit_e=True)
    s_n2 = _sc_segsum(h2h2.reshape(NC * N_NODES, 128), inter_e2n, zeros128)
    n_out = _tc_c2(s_n2, h2t2, hist_n, a2d)
    return (n_out, e_out)


# speedup vs baseline: 17.4277x; 1.0000x over previous
import functools

import jax
import jax.numpy as jnp
from jax import lax
from jax.experimental import pallas as pl
from jax.experimental.pallas import tpu as pltpu
from jax.experimental.pallas import tpu_sc as plsc

N_NODES = 10000
N_EDGES = 10000
E_PAIRS = 320000
NC = 2
NS = 16
PER_SUB = E_PAIRS // NS
BATCH = 128
FULL_BATCHES = PER_SUB // BATCH
REM = PER_SUB - FULL_BATCHES * BATCH
ROWS_PER_SUB = N_NODES // NS
GROUP = 12
GROUPS = 13
GROUP_P = 6
GROUPS_P = 13

_f32 = jnp.float32


def _prelu(v, a):
    return jnp.where(v >= 0, v, a * v)


def _sc_mesh():
    return plsc.VectorSubcoreMesh(core_axis_name="c", subcore_axis_name="s")


def _hist_body(hei_ref, ones_ref, zeros_ref, out_ref,
               idx2, idx_r, ones_v, ones_r, acc, sem_i):
    c = lax.axis_index("c")
    s = lax.axis_index("s")
    r0 = s * ROWS_PER_SUB
    pltpu.sync_copy(zeros_ref, acc.at[pl.ds(r0, ROWS_PER_SUB)])
    pltpu.sync_copy(ones_ref, ones_v)
    pltpu.sync_copy(ones_ref.at[pl.ds(0, REM)], ones_r)
    plsc.subcore_barrier()
    base = c * E_PAIRS + s * PER_SUB

    def idx_start(g, slot):
        pltpu.async_copy(hei_ref.at[pl.ds(base + g * BATCH, BATCH)],
                         idx2.at[slot], sem_i)

    def idx_wait(slot):
        pltpu.make_async_copy(hei_ref.at[pl.ds(0, BATCH)], idx2.at[slot],
                              sem_i).wait()

    idx_start(0, 0)
    idx_start(1, 1)

    def body(g, carry):
        p = lax.rem(g, 2)
        idx_wait(p)
        pltpu.sync_copy(ones_v, acc.at[idx2.at[p]], add=True)

        @pl.when(g + 2 < FULL_BATCHES)
        def _():
            idx_start(g + 2, p)

        return carry

    lax.fori_loop(0, FULL_BATCHES, body, 0)
    pltpu.sync_copy(hei_ref.at[pl.ds(base + FULL_BATCHES * BATCH, REM)], idx_r)
    pltpu.sync_copy(ones_r, acc.at[idx_r], add=True)
    plsc.subcore_barrier()
    pltpu.sync_copy(acc.at[pl.ds(r0, ROWS_PER_SUB)],
                    out_ref.at[c, pl.ds(r0, ROWS_PER_SUB)])


def _sc_hist(hei, ones16, zeros16):
    return pl.kernel(
        _hist_body,
        out_type=jax.ShapeDtypeStruct((NC, N_NODES, 16), _f32),
        mesh=_sc_mesh(),
        compiler_params=pltpu.CompilerParams(use_tc_tiling_on_sc=False),
        scratch_types=[
            pltpu.VMEM((2, BATCH), jnp.int32),
            pltpu.VMEM((REM,), jnp.int32),
            pltpu.VMEM((BATCH, 16), _f32),
            pltpu.VMEM((REM, 16), _f32),
            pltpu.VMEM_SHARED((N_NODES, 16), _f32),
            pltpu.SemaphoreType.DMA,
        ],
    )(hei, ones16, zeros16)


def _make_segsum_body(bpc, group, groups):
  def _segsum_body(h_ref, inter_ref, zeros_ref, out_ref,
                   ibuf, rows2, acc, sem_g, sem_i, sem_s):
      c = lax.axis_index("c")
      s = lax.axis_index("s")
      r0 = s * ROWS_PER_SUB
      pltpu.sync_copy(zeros_ref, acc.at[pl.ds(r0, ROWS_PER_SUB)])
      plsc.subcore_barrier()

      b_lo = (bpc * s) // 16
      b_hi = (bpc * (s + 1)) // 16
      leftover = b_hi - b_lo - groups * group
      row0 = c * bpc + b_lo

      def idx_start(grp, slot):
          pltpu.async_copy(inter_ref.at[pl.ds(row0 + grp * group, group)],
                           ibuf.at[slot], sem_i)

      def idx_wait(slot):
          pltpu.make_async_copy(inter_ref.at[pl.ds(0, group)], ibuf.at[slot],
                                sem_i).wait()

      def gath_start(slot, j, p):
          pltpu.async_copy(h_ref.at[ibuf.at[slot, j, 0]], rows2.at[p], sem_g)

      def gath_wait(p):
          pltpu.make_async_copy(h_ref.at[ibuf.at[0, 0, 0]], rows2.at[p],
                                sem_g).wait()

      def scat_wait():
          pltpu.make_async_copy(rows2.at[0], acc.at[ibuf.at[0, 0, 1]],
                                sem_s).wait()

      pltpu.sync_copy(inter_ref.at[pl.ds(row0, group)], ibuf.at[0])
      gath_start(0, 0, 0)

      def group_fn(grp, carry):
          slot = lax.rem(grp, 2)
          nslot = 1 - slot
          for j in range(group):
              m = grp * group + j
              pj = j % 2
              gath_wait(pj)

              @pl.when(m >= 1)
              def _():
                  scat_wait()

              if j == 0:
                  @pl.when(grp + 1 < groups)
                  def _():
                      idx_start(grp + 1, nslot)

              if j + 1 < group:
                  gath_start(slot, j + 1, 1 - pj)
              else:
                  @pl.when(grp + 1 < groups)
                  def _():
                      idx_wait(nslot)
                      gath_start(nslot, 0, 1 - pj)

              pltpu.async_copy(rows2.at[pj], acc.at[ibuf.at[slot, j, 1]], sem_s,
                               add=True)
          return carry

      lax.fori_loop(0, groups, group_fn, 0)
      scat_wait()

      @pl.when(leftover > 0)
      def _():
          pltpu.sync_copy(inter_ref.at[pl.ds(row0 + groups * group, 1)],
                          ibuf.at[0, pl.ds(0, 1)])
          pltpu.async_copy(h_ref.at[ibuf.at[0, 0, 0]], rows2.at[0], sem_g).wait()
          pltpu.sync_copy(rows2.at[0], acc.at[ibuf.at[0, 0, 1]], add=True)

      plsc.subcore_barrier()
      pltpu.sync_copy(acc.at[pl.ds(r0, ROWS_PER_SUB)],
                      out_ref.at[c, pl.ds(r0, ROWS_PER_SUB)])


  return _segsum_body


def _sc_segsum(h_flat, inter, zeros128):
    return pl.kernel(
        _make_segsum_body(2500, GROUP, GROUPS),
        out_type=jax.ShapeDtypeStruct((NC, N_NODES, 128), _f32),
        mesh=_sc_mesh(),
        compiler_params=pltpu.CompilerParams(use_tc_tiling_on_sc=False),
        scratch_types=[
            pltpu.VMEM((2, GROUP, 2, BATCH), jnp.int32),
            pltpu.VMEM((2, BATCH, 128), _f32),
            pltpu.VMEM_SHARED((N_NODES, 128), _f32),
            pltpu.SemaphoreType.DMA,
            pltpu.SemaphoreType.DMA,
            pltpu.SemaphoreType.DMA,
        ],
    )(h_flat, inter, zeros128)


def _sc_segsum_pairs(table, inter, zeros128):
    return pl.kernel(
        _make_segsum_body(1250, GROUP_P, GROUPS_P),
        out_type=jax.ShapeDtypeStruct((NC, N_NODES, 128), _f32),
        mesh=_sc_mesh(),
        compiler_params=pltpu.CompilerParams(use_tc_tiling_on_sc=False),
        scratch_types=[
            pltpu.VMEM((2, GROUP_P, 2, BATCH), jnp.int32),
            pltpu.VMEM((2, BATCH, 128), _f32),
            pltpu.VMEM_SHARED((N_NODES, 128), _f32),
            pltpu.SemaphoreType.DMA,
            pltpu.SemaphoreType.DMA,
            pltpu.SemaphoreType.DMA,
        ],
    )(table, inter, zeros128)


BM = 1000
GRID = N_NODES // BM


def _chunked_spec():
    return pl.BlockSpec((NC, BM, 128), lambda i: (0, i, 0))


def _w_spec(k):
    return pl.BlockSpec((k, 256), lambda i: (0, 0))


def _b_spec():
    return pl.BlockSpec((1, 256), lambda i: (0, 0))


def _hist_spec():
    return pl.BlockSpec((BM, 16), lambda i: (i, 0))


def _a_spec():
    return pl.BlockSpec((1, 1), lambda i: (0, 0), memory_space=pltpu.SMEM)


def _write_chunked(out_ref, v):
    out_ref[0] = v[:, :128]
    out_ref[1] = v[:, 128:]


def _cat(ref):
    return jnp.concatenate([ref[0], ref[1]], axis=1)


def _tc_b1_body(sx_ref, x_ref, hist_ref, wa_ref, ba_ref, w_ref, b_ref, a_ref,
                h2h_ref, h2t_ref):
    a = a_ref[0, 0]
    hist = hist_ref[:, 0:1]
    de_inv = jnp.where(hist > 0, 1.0 / hist, 0.0)
    sx = sx_ref[0] + sx_ref[1]
    esum = jnp.dot(sx, wa_ref[...], preferred_element_type=_f32) + hist * ba_ref[...]
    e_head = _prelu(de_inv * esum, a)
    h1 = jnp.dot(x_ref[...], wa_ref[...], preferred_element_type=_f32) + ba_ref[...]
    e_tail = _prelu(h1, a)
    h2h = jnp.dot(e_head, w_ref[...], preferred_element_type=_f32) + b_ref[...]
    h2t = jnp.dot(e_tail, w_ref[...], preferred_element_type=_f32) + b_ref[...]
    _write_chunked(h2h_ref, h2h)
    _write_chunked(h2t_ref, h2t)


def _tc_b1(sx, x, hist_e, Wa, ba2d, W, b2d, a2d):
    return pl.pallas_call(
        _tc_b1_body,
        grid=(GRID,),
        in_specs=[_chunked_spec(), pl.BlockSpec((BM, 128), lambda i: (i, 0)),
                  _hist_spec(), _w_spec(128), _b_spec(), _w_spec(256),
                  _b_spec(), _a_spec()],
        out_specs=[_chunked_spec(), _chunked_spec()],
        out_shape=[jax.ShapeDtypeStruct((NC, N_NODES, 128), _f32),
                   jax.ShapeDtypeStruct((NC, N_NODES, 128), _f32)],
    )(sx, x, hist_e, Wa, ba2d, W, b2d, a2d)


def _tc_b_body(emit_e, se_ref, h_ref, hist_ref, w_ref, b_ref, a_ref, *out_refs):
    a = a_ref[0, 0]
    hist = hist_ref[:, 0:1]
    de_inv = jnp.where(hist > 0, 1.0 / hist, 0.0)
    e_head = _prelu(de_inv * _cat(se_ref), a)
    e_tail = _prelu(_cat(h_ref), a)
    h2h = jnp.dot(e_head, w_ref[...], preferred_element_type=_f32) + b_ref[...]
    h2t = jnp.dot(e_tail, w_ref[...], preferred_element_type=_f32) + b_ref[...]
    _write_chunked(out_refs[0], h2h)
    _write_chunked(out_refs[1], h2t)
    if emit_e:
        out_refs[2][...] = e_head


def _tc_b(se, h, hist_e, W, b2d, a2d, emit_e):
    out_shapes = [jax.ShapeDtypeStruct((NC, N_NODES, 128), _f32),
                  jax.ShapeDtypeStruct((NC, N_NODES, 128), _f32)]
    out_specs = [_chunked_spec(), _chunked_spec()]
    if emit_e:
        out_shapes.append(jax.ShapeDtypeStruct((N_NODES, 256), _f32))
        out_specs.append(pl.BlockSpec((BM, 256), lambda i: (i, 0)))
    return pl.pallas_call(
        functools.partial(_tc_b_body, emit_e),
        grid=(GRID,),
        in_specs=[_chunked_spec(), _chunked_spec(), _hist_spec(),
                  _w_spec(256), _b_spec(), _a_spec()],
        out_specs=out_specs,
        out_shape=out_shapes,
    )(se, h, hist_e, W, b2d, a2d)


def _tc_c1_body(sn_ref, h2t_ref, hist_ref, w_ref, b_ref, a_ref, out_ref):
    a = a_ref[0, 0]
    dn_inv = 1.0 / (hist_ref[:, 0:1] + 1.0)
    n1 = _prelu(dn_inv * (_cat(sn_ref) + _cat(h2t_ref)), a)
    h = jnp.dot(n1, w_ref[...], preferred_element_type=_f32) + b_ref[...]
    _write_chunked(out_ref, h)


def _tc_c1(sn, h2t, hist_n, W, b2d, a2d):
    return pl.pallas_call(
        _tc_c1_body,
        grid=(GRID,),
        in_specs=[_chunked_spec(), _chunked_spec(), _hist_spec(),
                  _w_spec(256), _b_spec(), _a_spec()],
        out_specs=_chunked_spec(),
        out_shape=jax.ShapeDtypeStruct((NC, N_NODES, 128), _f32),
    )(sn, h2t, hist_n, W, b2d, a2d)


def _tc_c2_body(sn_ref, h2t_ref, hist_ref, a_ref, out_ref):
    a = a_ref[0, 0]
    dn_inv = 1.0 / (hist_ref[:, 0:1] + 1.0)
    out_ref[...] = _prelu(dn_inv * (_cat(sn_ref) + _cat(h2t_ref)), a)


def _tc_c2(sn, h2t, hist_n, a2d):
    return pl.pallas_call(
        _tc_c2_body,
        grid=(GRID,),
        in_specs=[_chunked_spec(), _chunked_spec(), _hist_spec(), _a_spec()],
        out_specs=pl.BlockSpec((BM, 256), lambda i: (i, 0)),
        out_shape=jax.ShapeDtypeStruct((N_NODES, 256), _f32),
    )(sn, h2t, hist_n, a2d)


def kernel(x, W1_n2e, b1_n2e, W1_e2n, b1_e2n, W2_n2e, b2_n2e, W2_e2n, b2_e2n,
           prelu_a, hyperedge_index, num_nodes, num_edges):
    del num_nodes, num_edges
    ni = hyperedge_index[0]
    ei = hyperedge_index[1]

    def make_inter(src, dst):
        sb = src.reshape(E_PAIRS // BATCH, BATCH)
        db = dst.reshape(E_PAIRS // BATCH, BATCH)
        return jnp.concatenate(
            [jnp.stack([sb, db], axis=1),
             jnp.stack([sb + N_NODES, db], axis=1)], axis=0)

    inter_n2e = make_inter(ni, ei)
    inter_e2n = make_inter(ei, ni)

    ones16 = jnp.ones((BATCH, 16), _f32)
    zeros16 = jnp.zeros((ROWS_PER_SUB, 16), _f32)
    zeros128 = jnp.zeros((ROWS_PER_SUB, 128), _f32)
    a2d = prelu_a.reshape(1, 1)

    hists = _sc_hist(hyperedge_index.reshape(2 * E_PAIRS), ones16, zeros16)
    hist_n = hists[0]
    hist_e = hists[1]

    s_x1 = _sc_segsum_pairs(x, inter_n2e, zeros128)
    h2h1, h2t1 = _tc_b1(s_x1, x, hist_e, W1_n2e, b1_n2e.reshape(1, 256),
                        W1_e2n, b1_e2n.reshape(1, 256), a2d)
    s_n1 = _sc_segsum(h2h1.reshape(NC * N_NODES, 128), inter_e2n, zeros128)
    hA2 = _tc_c1(s_n1, h2t1, hist_n, W2_n2e, b2_n2e.reshape(1, 256), a2d)
    s_e2 = _sc_segsum(hA2.reshape(NC * N_NODES, 128), inter_n2e, zeros128)
    h2h2, h2t2, e_out = _tc_b(s_e2, hA2, hist_e, W2_e2n, b2_e2n.reshape(1, 256),
                              a2d, emit_e=True)
    s_n2 = _sc_segsum(h2h2.reshape(NC * N_NODES, 128), inter_e2n, zeros128)
    n_out = _tc_c2(s_n2, h2t2, hist_n, a2d)
    return (n_out, e_out)


# speedup vs baseline: 17.4819x; 1.0031x over previous
import functools

import jax
import jax.numpy as jnp
from jax import lax
from jax.experimental import pallas as pl
from jax.experimental.pallas import tpu as pltpu
from jax.experimental.pallas import tpu_sc as plsc

N_NODES = 10000
N_EDGES = 10000
E_PAIRS = 320000
NC = 2
NS = 16
PER_SUB = E_PAIRS // NS
BATCH = 128
FULL_BATCHES = PER_SUB // BATCH
REM = PER_SUB - FULL_BATCHES * BATCH
ROWS_PER_SUB = N_NODES // NS
GROUP = 12
GROUPS = 13
GROUP_P = 6
GROUPS_P = 13

_f32 = jnp.float32


def _prelu(v, a):
    return jnp.where(v >= 0, v, a * v)


def _sc_mesh():
    return plsc.VectorSubcoreMesh(core_axis_name="c", subcore_axis_name="s")


def _hist_body(hei_ref, ones_ref, zeros_ref, out_ref,
               idx2, idx_r, ones_v, ones_r, acc, sem_i):
    c = lax.axis_index("c")
    s = lax.axis_index("s")
    r0 = s * ROWS_PER_SUB
    pltpu.sync_copy(zeros_ref, acc.at[pl.ds(r0, ROWS_PER_SUB)])
    pltpu.sync_copy(ones_ref, ones_v)
    pltpu.sync_copy(ones_ref.at[pl.ds(0, REM)], ones_r)
    plsc.subcore_barrier()
    base = c * E_PAIRS + s * PER_SUB

    def idx_start(g, slot):
        pltpu.async_copy(hei_ref.at[pl.ds(base + g * BATCH, BATCH)],
                         idx2.at[slot], sem_i)

    def idx_wait(slot):
        pltpu.make_async_copy(hei_ref.at[pl.ds(0, BATCH)], idx2.at[slot],
                              sem_i).wait()

    idx_start(0, 0)
    idx_start(1, 1)

    def body(g, carry):
        p = lax.rem(g, 2)
        idx_wait(p)
        pltpu.sync_copy(ones_v, acc.at[idx2.at[p]], add=True)

        @pl.when(g + 2 < FULL_BATCHES)
        def _():
            idx_start(g + 2, p)

        return carry

    lax.fori_loop(0, FULL_BATCHES, body, 0)
    pltpu.sync_copy(hei_ref.at[pl.ds(base + FULL_BATCHES * BATCH, REM)], idx_r)
    pltpu.sync_copy(ones_r, acc.at[idx_r], add=True)
    plsc.subcore_barrier()
    pltpu.sync_copy(acc.at[pl.ds(r0, ROWS_PER_SUB)],
                    out_ref.at[c, pl.ds(r0, ROWS_PER_SUB)])


def _sc_hist(hei, ones16, zeros16):
    return pl.kernel(
        _hist_body,
        out_type=jax.ShapeDtypeStruct((NC, N_NODES, 16), _f32),
        mesh=_sc_mesh(),
        compiler_params=pltpu.CompilerParams(use_tc_tiling_on_sc=False),
        scratch_types=[
            pltpu.VMEM((2, BATCH), jnp.int32),
            pltpu.VMEM((REM,), jnp.int32),
            pltpu.VMEM((BATCH, 16), _f32),
            pltpu.VMEM((REM, 16), _f32),
            pltpu.VMEM_SHARED((N_NODES, 16), _f32),
            pltpu.SemaphoreType.DMA,
        ],
    )(hei, ones16, zeros16)


def _make_segsum_body(bpc, group, groups):
  def _segsum_body(h_ref, inter_ref, zeros_ref, out_ref,
                   ibuf, rows2, acc, sem_g, sem_i, sem_s):
      c = lax.axis_index("c")
      s = lax.axis_index("s")
      r0 = s * ROWS_PER_SUB
      pltpu.sync_copy(zeros_ref, acc.at[pl.ds(r0, ROWS_PER_SUB)])
      plsc.subcore_barrier()

      b_lo = (bpc * s) // 16
      b_hi = (bpc * (s + 1)) // 16
      leftover = b_hi - b_lo - groups * group
      row0 = c * bpc + b_lo

      def idx_start(grp, slot):
          pltpu.async_copy(inter_ref.at[pl.ds(row0 + grp * group, group)],
                           ibuf.at[slot], sem_i)

      def idx_wait(slot):
          pltpu.make_async_copy(inter_ref.at[pl.ds(0, group)], ibuf.at[slot],
                                sem_i).wait()

      def gath_start(slot, j, p):
          pltpu.async_copy(h_ref.at[ibuf.at[slot, j, 0]], rows2.at[p], sem_g)

      def gath_wait(p):
          pltpu.make_async_copy(h_ref.at[ibuf.at[0, 0, 0]], rows2.at[p],
                                sem_g).wait()

      def scat_wait():
          pltpu.make_async_copy(rows2.at[0], acc.at[ibuf.at[0, 0, 1]],
                                sem_s).wait()

      pltpu.sync_copy(inter_ref.at[pl.ds(row0, group)], ibuf.at[0])
      gath_start(0, 0, 0)

      def group_fn(grp, carry):
          slot = lax.rem(grp, 2)
          nslot = 1 - slot
          for j in range(group):
              m = grp * group + j
              pj = j % 2
              gath_wait(pj)

              @pl.when(m >= 1)
              def _():
                  scat_wait()

              if j == 0:
                  @pl.when(grp + 1 < groups)
                  def _():
                      idx_start(grp + 1, nslot)

              if j + 1 < group:
                  gath_start(slot, j + 1, 1 - pj)
              else:
                  @pl.when(grp + 1 < groups)
                  def _():
                      idx_wait(nslot)
                      gath_start(nslot, 0, 1 - pj)

              pltpu.async_copy(rows2.at[pj], acc.at[ibuf.at[slot, j, 1]], sem_s,
                               add=True)
          return carry

      lax.fori_loop(0, groups, group_fn, 0)
      scat_wait()

      @pl.when(leftover > 0)
      def _():
          pltpu.sync_copy(inter_ref.at[pl.ds(row0 + groups * group, 1)],
                          ibuf.at[0, pl.ds(0, 1)])
          pltpu.async_copy(h_ref.at[ibuf.at[0, 0, 0]], rows2.at[0], sem_g).wait()
          pltpu.sync_copy(rows2.at[0], acc.at[ibuf.at[0, 0, 1]], add=True)

      plsc.subcore_barrier()
      pltpu.sync_copy(acc.at[pl.ds(r0, ROWS_PER_SUB)],
                      out_ref.at[c, pl.ds(r0, ROWS_PER_SUB)])


  return _segsum_body


def _sc_segsum(h_flat, inter, zeros128):
    return pl.kernel(
        _make_segsum_body(2500, GROUP, GROUPS),
        out_type=jax.ShapeDtypeStruct((NC, N_NODES, 128), _f32),
        mesh=_sc_mesh(),
        compiler_params=pltpu.CompilerParams(use_tc_tiling_on_sc=False),
        scratch_types=[
            pltpu.VMEM((2, GROUP, 2, BATCH), jnp.int32),
            pltpu.VMEM((2, BATCH, 128), _f32),
            pltpu.VMEM_SHARED((N_NODES, 128), _f32),
            pltpu.SemaphoreType.DMA,
            pltpu.SemaphoreType.DMA,
            pltpu.SemaphoreType.DMA,
        ],
    )(h_flat, inter, zeros128)


def _sc_segsum_pairs(table, inter, zeros128):
    return pl.kernel(
        _make_segsum_body(1250, GROUP_P, GROUPS_P),
        out_type=jax.ShapeDtypeStruct((NC, N_NODES, 128), _f32),
        mesh=_sc_mesh(),
        compiler_params=pltpu.CompilerParams(use_tc_tiling_on_sc=False),
        scratch_types=[
            pltpu.VMEM((2, GROUP_P, 2, BATCH), jnp.int32),
            pltpu.VMEM((2, BATCH, 128), _f32),
            pltpu.VMEM_SHARED((N_NODES, 128), _f32),
            pltpu.SemaphoreType.DMA,
            pltpu.SemaphoreType.DMA,
            pltpu.SemaphoreType.DMA,
        ],
    )(table, inter, zeros128)


BM = 2000
GRID = N_NODES // BM


def _chunked_spec():
    return pl.BlockSpec((NC, BM, 128), lambda i: (0, i, 0))


def _w_spec(k):
    return pl.BlockSpec((k, 256), lambda i: (0, 0))


def _b_spec():
    return pl.BlockSpec((1, 256), lambda i: (0, 0))


def _hist_spec():
    return pl.BlockSpec((BM, 16), lambda i: (i, 0))


def _a_spec():
    return pl.BlockSpec((1, 1), lambda i: (0, 0), memory_space=pltpu.SMEM)


def _write_chunked(out_ref, v):
    out_ref[0] = v[:, :128]
    out_ref[1] = v[:, 128:]


def _cat(ref):
    return jnp.concatenate([ref[0], ref[1]], axis=1)


def _tc_b1_body(sx_ref, x_ref, hist_ref, wa_ref, ba_ref, w_ref, b_ref, a_ref,
                h2h_ref, h2t_ref):
    a = a_ref[0, 0]
    hist = hist_ref[:, 0:1]
    de_inv = jnp.where(hist > 0, 1.0 / hist, 0.0)
    sx = sx_ref[0] + sx_ref[1]
    esum = jnp.dot(sx, wa_ref[...], preferred_element_type=_f32) + hist * ba_ref[...]
    e_head = _prelu(de_inv * esum, a)
    h1 = jnp.dot(x_ref[...], wa_ref[...], preferred_element_type=_f32) + ba_ref[...]
    e_tail = _prelu(h1, a)
    h2h = jnp.dot(e_head, w_ref[...], preferred_element_type=_f32) + b_ref[...]
    h2t = jnp.dot(e_tail, w_ref[...], preferred_element_type=_f32) + b_ref[...]
    _write_chunked(h2h_ref, h2h)
    _write_chunked(h2t_ref, h2t)


def _tc_b1(sx, x, hist_e, Wa, ba2d, W, b2d, a2d):
    return pl.pallas_call(
        _tc_b1_body,
        grid=(GRID,),
        in_specs=[_chunked_spec(), pl.BlockSpec((BM, 128), lambda i: (i, 0)),
                  _hist_spec(), _w_spec(128), _b_spec(), _w_spec(256),
                  _b_spec(), _a_spec()],
        out_specs=[_chunked_spec(), _chunked_spec()],
        out_shape=[jax.ShapeDtypeStruct((NC, N_NODES, 128), _f32),
                   jax.ShapeDtypeStruct((NC, N_NODES, 128), _f32)],
    )(sx, x, hist_e, Wa, ba2d, W, b2d, a2d)


def _tc_b_body(emit_e, se_ref, h_ref, hist_ref, w_ref, b_ref, a_ref, *out_refs):
    a = a_ref[0, 0]
    hist = hist_ref[:, 0:1]
    de_inv = jnp.where(hist > 0, 1.0 / hist, 0.0)
    e_head = _prelu(de_inv * _cat(se_ref), a)
    e_tail = _prelu(_cat(h_ref), a)
    h2h = jnp.dot(e_head, w_ref[...], preferred_element_type=_f32) + b_ref[...]
    h2t = jnp.dot(e_tail, w_ref[...], preferred_element_type=_f32) + b_ref[...]
    _write_chunked(out_refs[0], h2h)
    _write_chunked(out_refs[1], h2t)
    if emit_e:
        out_refs[2][...] = e_head


def _tc_b(se, h, hist_e, W, b2d, a2d, emit_e):
    out_shapes = [jax.ShapeDtypeStruct((NC, N_NODES, 128), _f32),
                  jax.ShapeDtypeStruct((NC, N_NODES, 128), _f32)]
    out_specs = [_chunked_spec(), _chunked_spec()]
    if emit_e:
        out_shapes.append(jax.ShapeDtypeStruct((N_NODES, 256), _f32))
        out_specs.append(pl.BlockSpec((BM, 256), lambda i: (i, 0)))
    return pl.pallas_call(
        functools.partial(_tc_b_body, emit_e),
        grid=(GRID,),
        in_specs=[_chunked_spec(), _chunked_spec(), _hist_spec(),
                  _w_spec(256), _b_spec(), _a_spec()],
        out_specs=out_specs,
        out_shape=out_shapes,
    )(se, h, hist_e, W, b2d, a2d)


def _tc_c1_body(sn_ref, h2t_ref, hist_ref, w_ref, b_ref, a_ref, out_ref):
    a = a_ref[0, 0]
    dn_inv = 1.0 / (hist_ref[:, 0:1] + 1.0)
    n1 = _prelu(dn_inv * (_cat(sn_ref) + _cat(h2t_ref)), a)
    h = jnp.dot(n1, w_ref[...], preferred_element_type=_f32) + b_ref[...]
    _write_chunked(out_ref, h)


def _tc_c1(sn, h2t, hist_n, W, b2d, a2d):
    return pl.pallas_call(
        _tc_c1_body,
        grid=(GRID,),
        in_specs=[_chunked_spec(), _chunked_spec(), _hist_spec(),
                  _w_spec(256), _b_spec(), _a_spec()],
        out_specs=_chunked_spec(),
        out_shape=jax.ShapeDtypeStruct((NC, N_NODES, 128), _f32),
    )(sn, h2t, hist_n, W, b2d, a2d)


def _tc_c2_body(sn_ref, h2t_ref, hist_ref, a_ref, out_ref):
    a = a_ref[0, 0]
    dn_inv = 1.0 / (hist_ref[:, 0:1] + 1.0)
    out_ref[...] = _prelu(dn_inv * (_cat(sn_ref) + _cat(h2t_ref)), a)


def _tc_c2(sn, h2t, hist_n, a2d):
    return pl.pallas_call(
        _tc_c2_body,
        grid=(GRID,),
        in_specs=[_chunked_spec(), _chunked_spec(), _hist_spec(), _a_spec()],
        out_specs=pl.BlockSpec((BM, 256), lambda i: (i, 0)),
        out_shape=jax.ShapeDtypeStruct((N_NODES, 256), _f32),
    )(sn, h2t, hist_n, a2d)


def kernel(x, W1_n2e, b1_n2e, W1_e2n, b1_e2n, W2_n2e, b2_n2e, W2_e2n, b2_e2n,
           prelu_a, hyperedge_index, num_nodes, num_edges):
    del num_nodes, num_edges
    ni = hyperedge_index[0]
    ei = hyperedge_index[1]

    def make_inter(src, dst):
        sb = src.reshape(E_PAIRS // BATCH, BATCH)
        db = dst.reshape(E_PAIRS // BATCH, BATCH)
        return jnp.concatenate(
            [jnp.stack([sb, db], axis=1),
             jnp.stack([sb + N_NODES, db], axis=1)], axis=0)

    inter_n2e = make_inter(ni, ei)
    inter_e2n = make_inter(ei, ni)

    ones16 = jnp.ones((BATCH, 16), _f32)
    zeros16 = jnp.zeros((ROWS_PER_SUB, 16), _f32)
    zeros128 = jnp.zeros((ROWS_PER_SUB, 128), _f32)
    a2d = prelu_a.reshape(1, 1)

    hists = _sc_hist(hyperedge_index.reshape(2 * E_PAIRS), ones16, zeros16)
    hist_n = hists[0]
    hist_e = hists[1]

    s_x1 = _sc_segsum_pairs(x, inter_n2e, zeros128)
    h2h1, h2t1 = _tc_b1(s_x1, x, hist_e, W1_n2e, b1_n2e.reshape(1, 256),
                        W1_e2n, b1_e2n.reshape(1, 256), a2d)
    s_n1 = _sc_segsum(h2h1.reshape(NC * N_NODES, 128), inter_e2n, zeros128)
    hA2 = _tc_c1(s_n1, h2t1, hist_n, W2_n2e, b2_n2e.reshape(1, 256), a2d)
    s_e2 = _sc_segsum(hA2.reshape(NC * N_NODES, 128), inter_n2e, zeros128)
    h2h2, h2t2, e_out = _tc_b(s_e2, hA2, hist_e, W2_e2n, b2_e2n.reshape(1, 256),
                              a2d, emit_e=True)
    s_n2 = _sc_segsum(h2h2.reshape(NC * N_NODES, 128), inter_e2n, zeros128)
    n_out = _tc_c2(s_n2, h2t2, hist_n, a2d)
    return (n_out, e_out)
